# Initial kernel scaffold; baseline (speedup 1.0000x reference)
import functools

import jax
import jax.numpy as jnp
from jax import lax
from jax.experimental import pallas as pl
from jax.experimental.pallas import tpu as pltpu
from jax.experimental.pallas import tpu_sc as plsc

NC = 2
NS = 16
NW = NC * NS
LANES = 16

F32 = jnp.float32


def _mesh():
    return plsc.VectorSubcoreMesh(
        core_axis_name="c", subcore_axis_name="s", num_cores=NC, num_subcores=NS
    )


def _chunk_1d(n):
    for k in range(128, 0, -8):
        if n % k == 0:
            return k
    raise ValueError(f"no 8-aligned chunk for {n}")


def _chunk_rows(n):
    for k in range(128, 0, -1):
        if n % k == 0:
            return k
    raise ValueError(n)


def _wid():
    return lax.axis_index("s") * NC + lax.axis_index("c")


@functools.lru_cache(maxsize=None)
def _edge_kernel(E, N, D):
    EPW = E // NW
    EK = _chunk_1d(EPW)
    NCH = EPW // EK
    NPT = N // NS
    ZC = _chunk_rows(NPT)
    NZ = NPT // ZC
    DK = D // LANES
    inv_scale = 1.0 / float(D) ** 0.5

    @functools.partial(
        pl.kernel,
        out_type=(
            jax.ShapeDtypeStruct((NC * N, D), F32),
            jax.ShapeDtypeStruct((NC * N, LANES), F32),
        ),
        mesh=_mesh(),
        scratch_types=[
            pltpu.VMEM((EK, D), F32),
            pltpu.VMEM((EK, D), F32),
            pltpu.VMEM((EK, D), F32),
            pltpu.VMEM((EK, LANES), F32),
            pltpu.VMEM((EK, LANES), F32),
            pltpu.VMEM((EK,), jnp.int32),
            pltpu.VMEM((EK,), jnp.int32),
            pltpu.VMEM((ZC, D), F32),
            pltpu.VMEM((ZC, LANES), F32),
            pltpu.VMEM_SHARED((N, D), F32),
            pltpu.VMEM_SHARED((N, LANES), F32),
            pltpu.SemaphoreType.DMA,
            pltpu.SemaphoreType.DMA,
        ],
    )
    def k1(edge_hbm, snd_hbm, rcv_hbm, nemb_hbm, u_out, den_out,
           e_v, nr_v, ns_v, exr_v, exs_v, ir_v, is_v, z_v, zd_v,
           u_sh, den_sh, sem_r, sem_s):
        sid = lax.axis_index("s")
        cid = lax.axis_index("c")
        wid = sid * NC + cid
        zero16 = jnp.zeros((LANES,), F32)

        def zrow(i, _):
            for j in range(DK):
                z_v[i, pl.ds(j * LANES, LANES)] = zero16
            zd_v[i, :] = zero16
            return 0

        lax.fori_loop(0, ZC, zrow, 0)
        for t in range(NZ):
            off = sid * NPT + t * ZC
            pltpu.sync_copy(z_v, u_sh.at[pl.ds(off, ZC)])
            pltpu.sync_copy(zd_v, den_sh.at[pl.ds(off, ZC)])
        plsc.subcore_barrier()

        col0 = (lax.iota(jnp.int32, LANES) == 0).astype(F32)
        ebase = wid * EPW

        def chunk(c, _):
            b = ebase + c * EK
            pltpu.sync_copy(snd_hbm.at[pl.ds(b, EK)], is_v)
            pltpu.sync_copy(rcv_hbm.at[pl.ds(b, EK)], ir_v)
            pltpu.sync_copy(edge_hbm.at[pl.ds(b, EK)], e_v)
            cr = pltpu.async_copy(nemb_hbm.at[ir_v], nr_v, sem_r)
            cs = pltpu.async_copy(nemb_hbm.at[is_v], ns_v, sem_s)
            cr.wait()
            cs.wait()

            def edge(i, _):
                es = [e_v[i, pl.ds(j * LANES, LANES)] for j in range(DK)]
                accr = es[0] * nr_v[i, pl.ds(0, LANES)]
                accs = es[0] * ns_v[i, pl.ds(0, LANES)]
                for j in range(1, DK):
                    accr = accr + es[j] * nr_v[i, pl.ds(j * LANES, LANES)]
                    accs = accs + es[j] * ns_v[i, pl.ds(j * LANES, LANES)]
                exr = jnp.exp(jnp.full((LANES,), jnp.sum(accr) * inv_scale, F32))
                exs = jnp.exp(jnp.full((LANES,), jnp.sum(accs) * inv_scale, F32))
                for j in range(DK):
                    nr_v[i, pl.ds(j * LANES, LANES)] = es[j] * exr
                    ns_v[i, pl.ds(j * LANES, LANES)] = es[j] * exs
                exr_v[i, :] = exr * col0
                exs_v[i, :] = exs * col0
                return 0

            lax.fori_loop(0, EK, edge, 0)
            pltpu.sync_copy(nr_v, u_sh.at[ir_v], add=True)
            pltpu.sync_copy(ns_v, u_sh.at[is_v], add=True)
            pltpu.sync_copy(exr_v, den_sh.at[ir_v], add=True)
            pltpu.sync_copy(exs_v, den_sh.at[is_v], add=True)
            return 0

        lax.fori_loop(0, NCH, chunk, 0)
        plsc.subcore_barrier()
        off = sid * NPT
        pltpu.sync_copy(u_sh.at[pl.ds(off, NPT)],
                        u_out.at[pl.ds(cid * N + off, NPT)])
        pltpu.sync_copy(den_sh.at[pl.ds(off, NPT)],
                        den_out.at[pl.ds(cid * N + off, NPT)])

    return k1


@functools.lru_cache(maxsize=None)
def _norm_kernel(N, D, softmax_eps):
    RW = (N // NW) // 8 * 8
    REM = N - RW * NW
    DK = D // LANES

    @functools.partial(
        pl.kernel,
        out_type=jax.ShapeDtypeStruct((N, D), F32),
        mesh=_mesh(),
        scratch_types=[
            pltpu.VMEM((RW, D), F32),
            pltpu.VMEM((RW, D), F32),
            pltpu.VMEM((RW, LANES), F32),
            pltpu.VMEM((RW, LANES), F32),
        ],
    )
    def k(x_hbm, d_hbm, o_hbm, a_v, b_v, da_v, db_v):
        wid = _wid()

        def do(base, n):
            pltpu.sync_copy(x_hbm.at[pl.ds(base, n)], a_v.at[pl.ds(0, n)])
            pltpu.sync_copy(x_hbm.at[pl.ds(N + base, n)], b_v.at[pl.ds(0, n)])
            pltpu.sync_copy(d_hbm.at[pl.ds(base, n)], da_v.at[pl.ds(0, n)])
            pltpu.sync_copy(d_hbm.at[pl.ds(N + base, n)], db_v.at[pl.ds(0, n)])

            def row(i, _):
                d = jnp.sum(da_v[i, :]) + jnp.sum(db_v[i, :])
                if softmax_eps:
                    inv = 1.0 / (d + 1e-16)
                else:
                    inv = 1.0 / jnp.maximum(d, 1.0)
                for j in range(DK):
                    sl = pl.ds(j * LANES, LANES)
                    a_v[i, sl] = (a_v[i, sl] + b_v[i, sl]) * inv
                return 0

            lax.fori_loop(0, n, row, 0)
            pltpu.sync_copy(a_v.at[pl.ds(0, n)], o_hbm.at[pl.ds(base, n)])

        do(wid * RW, RW)
        if REM:
            @pl.when(wid == NW - 1)
            def _():
                do(NW * RW, REM)

    return k


@functools.lru_cache(maxsize=None)
def _cell_gather_kernel(NCELL, N, D):
    CPW = (NCELL // NW) // 8 * 8
    REM = NCELL - CPW * NW
    CK = _chunk_1d(CPW)
    NCH = CPW // CK
    DK = D // LANES

    scratch = [
        pltpu.VMEM((CK, D), F32),
        pltpu.VMEM((CK, D), F32),
        pltpu.VMEM((CK, D), F32),
        pltpu.VMEM((CK, D), F32),
        pltpu.VMEM((CK,), jnp.int32),
        pltpu.VMEM((CK,), jnp.int32),
        pltpu.VMEM((CK,), jnp.int32),
        pltpu.SemaphoreType.DMA,
        pltpu.SemaphoreType.DMA,
        pltpu.SemaphoreType.DMA,
    ]
    if REM:
        scratch += [
            pltpu.VMEM((REM,), jnp.int32),
            pltpu.VMEM((REM,), jnp.int32),
            pltpu.VMEM((REM,), jnp.int32),
        ]

    @functools.partial(
        pl.kernel,
        out_type=jax.ShapeDtypeStruct((NCELL, D), F32),
        mesh=_mesh(),
        scratch_types=scratch,
    )
    def k(f0_hbm, f1_hbm, f2_hbm, nagg_hbm, o_hbm,
          g0_v, g1_v, g2_v, o_v, i0_v, i1_v, i2_v, s0, s1, s2, *tails):
        wid = _wid()

        def do(b, n, i0, i1, i2):
            pltpu.sync_copy(f0_hbm.at[pl.ds(b, n)], i0)
            pltpu.sync_copy(f1_hbm.at[pl.ds(b, n)], i1)
            pltpu.sync_copy(f2_hbm.at[pl.ds(b, n)], i2)
            c0 = pltpu.async_copy(nagg_hbm.at[i0], g0_v.at[pl.ds(0, n)], s0)
            c1 = pltpu.async_copy(nagg_hbm.at[i1], g1_v.at[pl.ds(0, n)], s1)
            c2 = pltpu.async_copy(nagg_hbm.at[i2], g2_v.at[pl.ds(0, n)], s2)
            c0.wait()
            c1.wait()
            c2.wait()

            def cell(i, _):
                for j in range(DK):
                    sl = pl.ds(j * LANES, LANES)
                    o_v[i, sl] = (g0_v[i, sl] + g1_v[i, sl] + g2_v[i, sl]) * (1.0 / 3.0)
                return 0

            lax.fori_loop(0, n, cell, 0)
            pltpu.sync_copy(o_v.at[pl.ds(0, n)], o_hbm.at[pl.ds(b, n)])

        def chunk(c, _):
            do(wid * CPW + c * CK, CK, i0_v, i1_v, i2_v)
            return 0

        lax.fori_loop(0, NCH, chunk, 0)
        if REM:
            t0, t1, t2 = tails

            @pl.when(wid == NW - 1)
            def _():
                do(NW * CPW, REM, t0, t1, t2)

    return k


@functools.lru_cache(maxsize=None)
def _cell_scatter_kernel(NCELL, N, D):
    CPW = (NCELL // NW) // 8 * 8
    REM = NCELL - CPW * NW
    CK = _chunk_1d(CPW)
    NCH = CPW // CK
    DK = D // LANES
    NPT = N // NS
    ZC = _chunk_rows(NPT)
    NZ = NPT // ZC

    scratch = [
        pltpu.VMEM((CK, D), F32),
        pltpu.VMEM((CK, LANES), F32),
        pltpu.VMEM((CK,), jnp.int32),
        pltpu.VMEM((CK,), jnp.int32),
        pltpu.VMEM((CK,), jnp.int32),
        pltpu.VMEM((ZC, D), F32),
        pltpu.VMEM((ZC, LANES), F32),
        pltpu.VMEM_SHARED((N, D), F32),
        pltpu.VMEM_SHARED((N, LANES), F32),
    ]
    if REM:
        scratch += [
            pltpu.VMEM((REM,), jnp.int32),
            pltpu.VMEM((REM,), jnp.int32),
            pltpu.VMEM((REM,), jnp.int32),
        ]

    @functools.partial(
        pl.kernel,
        out_type=(
            jax.ShapeDtypeStruct((NC * N, D), F32),
            jax.ShapeDtypeStruct((NC * N, LANES), F32),
        ),
        mesh=_mesh(),
        scratch_types=scratch,
    )
    def k(cell_hbm, f0_hbm, f1_hbm, f2_hbm, s_out, c_out,
          buf_v, one_v, i0_v, i1_v, i2_v, z_v, zc_v, s_sh, c_sh, *tails):
        sid = lax.axis_index("s")
        cid = lax.axis_index("c")
        wid = sid * NC + cid
        zero16 = jnp.zeros((LANES,), F32)
        col0 = (lax.iota(jnp.int32, LANES) == 0).astype(F32)

        def zrow(i, _):
            for j in range(DK):
                z_v[i, pl.ds(j * LANES, LANES)] = zero16
            zc_v[i, :] = zero16
            return 0

        lax.fori_loop(0, ZC, zrow, 0)

        def onerow(i, _):
            one_v[i, :] = col0
            return 0

        lax.fori_loop(0, CK, onerow, 0)
        for t in range(NZ):
            off = sid * NPT + t * ZC
            pltpu.sync_copy(z_v, s_sh.at[pl.ds(off, ZC)])
            pltpu.sync_copy(zc_v, c_sh.at[pl.ds(off, ZC)])
        plsc.subcore_barrier()

        def do(b, n, i0, i1, i2):
            pltpu.sync_copy(cell_hbm.at[pl.ds(b, n)], buf_v.at[pl.ds(0, n)])
            pltpu.sync_copy(f0_hbm.at[pl.ds(b, n)], i0)
            pltpu.sync_copy(f1_hbm.at[pl.ds(b, n)], i1)
            pltpu.sync_copy(f2_hbm.at[pl.ds(b, n)], i2)
            pltpu.sync_copy(buf_v.at[pl.ds(0, n)], s_sh.at[i0], add=True)
            pltpu.sync_copy(buf_v.at[pl.ds(0, n)], s_sh.at[i1], add=True)
            pltpu.sync_copy(buf_v.at[pl.ds(0, n)], s_sh.at[i2], add=True)
            pltpu.sync_copy(one_v.at[pl.ds(0, n)], c_sh.at[i0], add=True)
            pltpu.sync_copy(one_v.at[pl.ds(0, n)], c_sh.at[i1], add=True)
            pltpu.sync_copy(one_v.at[pl.ds(0, n)], c_sh.at[i2], add=True)

        def chunk(c, _):
            do(wid * CPW + c * CK, CK, i0_v, i1_v, i2_v)
            return 0

        lax.fori_loop(0, NCH, chunk, 0)
        if REM:
            t0, t1, t2 = tails

            @pl.when(wid == NW - 1)
            def _():
                do(NW * CPW, REM, t0, t1, t2)

        plsc.subcore_barrier()
        off = sid * NPT
        pltpu.sync_copy(s_sh.at[pl.ds(off, NPT)],
                        s_out.at[pl.ds(cid * N + off, NPT)])
        pltpu.sync_copy(c_sh.at[pl.ds(off, NPT)],
                        c_out.at[pl.ds(cid * N + off, NPT)])

    return k


def _matmul(cell_attr, cell_agg, w1, w2, b8):
    M, D = cell_attr.shape
    BM = 2500
    assert M % BM == 0

    def body(a_ref, g_ref, w1_ref, w2_ref, b_ref, o_ref):
        o_ref[...] = (
            jnp.dot(a_ref[...], w1_ref[...], preferred_element_type=F32)
            + jnp.dot(g_ref[...], w2_ref[...], preferred_element_type=F32)
            + b_ref[0:1, :]
        )

    return pl.pallas_call(
        body,
        grid=(M // BM,),
        in_specs=[
            pl.BlockSpec((BM, D), lambda i: (i, 0)),
            pl.BlockSpec((BM, D), lambda i: (i, 0)),
            pl.BlockSpec((D, D), lambda i: (0, 0)),
            pl.BlockSpec((D, D), lambda i: (0, 0)),
            pl.BlockSpec((8, D), lambda i: (0, 0)),
        ],
        out_specs=pl.BlockSpec((BM, D), lambda i: (i, 0)),
        out_shape=jax.ShapeDtypeStruct((M, D), F32),
    )(cell_attr, cell_agg, w1, w2, b8)


def kernel(cell_attr, edge_attr, node_embedding, edge_index, face, W, b):
    E, D = edge_attr.shape
    N = node_embedding.shape[0]
    NCELL = cell_attr.shape[0]

    senders = edge_index[0]
    receivers = edge_index[1]
    f0, f1, f2 = face[0], face[1], face[2]

    u_parts, den_parts = _edge_kernel(E, N, D)(
        edge_attr, senders, receivers, node_embedding)
    node_agg = _norm_kernel(N, D, True)(u_parts, den_parts)
    cell_agg = _cell_gather_kernel(NCELL, N, D)(f0, f1, f2, node_agg)
    b8 = jnp.broadcast_to(b.reshape(1, D), (8, D))
    cell_new = _matmul(cell_attr, cell_agg, W[:D], W[D:], b8)
    s_parts, c_parts = _cell_scatter_kernel(NCELL, N, D)(cell_new, f0, f1, f2)
    node_attr = _norm_kernel(N, D, False)(s_parts, c_parts)
    return cell_new, node_attr


# speedup vs baseline: 10.5175x; 10.5175x over previous
import functools

import jax
import jax.numpy as jnp
from jax import lax
from jax.experimental import pallas as pl
from jax.experimental.pallas import tpu as pltpu
from jax.experimental.pallas import tpu_sc as plsc

NC = 2
NS = 16
NW = NC * NS
LANES = 16

F32 = jnp.float32

def _mesh():
    return plsc.VectorSubcoreMesh(
        core_axis_name="c", subcore_axis_name="s", num_cores=NC, num_subcores=NS
    )


def _chunk_1d(n, cap=128):
    for k in range(cap, 0, -8):
        if n % k == 0:
            return k
    raise ValueError(f"no 8-aligned chunk for {n}")


def _hsum(x):
    iota = lax.iota(jnp.int32, LANES)
    for m in (1, 2, 4, 8):
        x = x + x.at[iota ^ m].get(mode="promise_in_bounds", unique_indices=True)
    return x


def _windows(n):
    wins = [(s, 0) for s in range(0, n - 15, 16)]
    cov = 16 * len(wins)
    if cov < n:
        wins.append((n - 16, 16 - (n - cov)))
    return wins


def _tile_reduce_dump(src_sh, acc_v, tmp_v, ta_v, tt_v, out_hbm, N, DPT, DREM,
                      sid, cid):
    zoff = sid * DPT
    pltpu.sync_copy(src_sh.at[pl.ds(zoff, DPT)], acc_v)
    for w in range(1, NS):
        pltpu.sync_copy(src_sh.at[pl.ds(w * N + zoff, DPT)], tmp_v)

        def addw(i, _):
            acc_v[pl.ds(i * 16, 16)] = (acc_v[pl.ds(i * 16, 16)]
                                        + tmp_v[pl.ds(i * 16, 16)])
            return 0

        lax.fori_loop(0, DPT // 16, addw, 0)
    pltpu.sync_copy(acc_v, out_hbm.at[pl.ds(cid * N + zoff, DPT)])
    if DREM:
        @pl.when(sid == NS - 1)
        def _():
            pltpu.sync_copy(src_sh.at[pl.ds(NS * DPT, DREM)], ta_v)
            for w in range(1, NS):
                pltpu.sync_copy(src_sh.at[pl.ds(w * N + NS * DPT, DREM)], tt_v)
                ta_v[...] = ta_v[...] + tt_v[...]
            pltpu.sync_copy(ta_v, out_hbm.at[pl.ds(cid * N + NS * DPT, DREM)])


@functools.lru_cache(maxsize=None)
def _edge_kernel(E, N, D):
    EPW = E // NW
    EK = _chunk_1d(EPW, cap=40)
    NCH = EPW // EK
    DPT = (N // NS) // 8 * 8
    DREM = N - DPT * NS
    DK = D // LANES
    inv_scale = 1.0 / float(D) ** 0.5
    assert N % 16 == 0

    @functools.partial(
        pl.kernel,
        out_type=(
            jax.ShapeDtypeStruct((NC * N, D), F32),
            jax.ShapeDtypeStruct((NC * N,), F32),
        ),
        mesh=_mesh(),
        scratch_types=[
            pltpu.VMEM((EK, D), F32),
            pltpu.VMEM((EK, D), F32),
            pltpu.VMEM((EK, D), F32),
            pltpu.VMEM((EK,), jnp.int32),
            pltpu.VMEM((EK,), jnp.int32),
            pltpu.VMEM((EK + 16,), jnp.int32),
            pltpu.VMEM((EK + 16,), jnp.int32),
            pltpu.VMEM((N + 16,), F32),
            pltpu.VMEM((DPT,), F32),
            pltpu.VMEM((DPT,), F32),
            pltpu.VMEM((16,), F32),
            pltpu.VMEM((16,), F32),
            pltpu.VMEM_SHARED((N, D), F32),
            pltpu.VMEM_SHARED((NS * N,), F32),
            pltpu.SemaphoreType.DMA,
            pltpu.SemaphoreType.DMA,
        ],
    )
    def k1(edge_hbm, snd_hbm, rcv_hbm, nemb_hbm, zn_hbm, u_out, den_out,
           e_v, nr_v, ns_v, ir_v, is_v, irp_v, isp_v, den_v, acc_v, tmp_v,
           ta_v, tt_v, u_sh, den_st, sem_r, sem_s):
        sid = lax.axis_index("s")
        cid = lax.axis_index("c")
        wid = sid * NC + cid
        iota = lax.iota(jnp.int32, LANES)
        zeros16 = jnp.zeros((LANES,), F32)

        def zden(i, _):
            den_v[pl.ds(i * 16, 16)] = zeros16
            return 0

        lax.fori_loop(0, (N + 16) // 16, zden, 0)

        zoff = sid * DPT
        pltpu.sync_copy(zn_hbm.at[pl.ds(zoff, DPT)], u_sh.at[pl.ds(zoff, DPT)])
        if DREM:
            @pl.when(sid == NS - 1)
            def _():
                pltpu.sync_copy(zn_hbm.at[pl.ds(NS * DPT, DREM)],
                                u_sh.at[pl.ds(NS * DPT, DREM)])
        plsc.subcore_barrier()

        inv_scale_v = jnp.full((LANES,), inv_scale, F32)
        onev = jnp.full((LANES,), 1.0, F32)
        eye0 = jnp.where(iota == 0, onev, zeros16)
        ebase = wid * EPW

        def chunk(c, _):
            b = ebase + c * EK
            pltpu.sync_copy(snd_hbm.at[pl.ds(b, EK)], is_v)
            pltpu.sync_copy(rcv_hbm.at[pl.ds(b, EK)], ir_v)
            pltpu.sync_copy(snd_hbm.at[pl.ds(b, EK)], isp_v.at[pl.ds(0, EK)])
            pltpu.sync_copy(rcv_hbm.at[pl.ds(b, EK)], irp_v.at[pl.ds(0, EK)])
            pltpu.sync_copy(edge_hbm.at[pl.ds(b, EK)], e_v)
            cr = pltpu.async_copy(nemb_hbm.at[ir_v], nr_v, sem_r)
            cs = pltpu.async_copy(nemb_hbm.at[is_v], ns_v, sem_s)
            cr.wait()
            cs.wait()

            def edge(i, _):
                es = [e_v[i, pl.ds(j * LANES, LANES)] for j in range(DK)]
                accr = es[0] * nr_v[i, pl.ds(0, LANES)]
                accs = es[0] * ns_v[i, pl.ds(0, LANES)]
                for j in range(1, DK):
                    accr = accr + es[j] * nr_v[i, pl.ds(j * LANES, LANES)]
                    accs = accs + es[j] * ns_v[i, pl.ds(j * LANES, LANES)]
                exr = jnp.exp(_hsum(accr) * inv_scale_v)
                exs = jnp.exp(_hsum(accs) * inv_scale_v)
                for j in range(DK):
                    nr_v[i, pl.ds(j * LANES, LANES)] = es[j] * exr
                    ns_v[i, pl.ds(j * LANES, LANES)] = es[j] * exs
                nri = irp_v[pl.ds(i, 16)][0]
                den_v[pl.ds(nri, 16)] = den_v[pl.ds(nri, 16)] + exr * eye0
                nsi = isp_v[pl.ds(i, 16)][0]
                den_v[pl.ds(nsi, 16)] = den_v[pl.ds(nsi, 16)] + exs * eye0
                return 0

            lax.fori_loop(0, EK, edge, 0)

            pltpu.sync_copy(nr_v, u_sh.at[ir_v], add=True)
            pltpu.sync_copy(ns_v, u_sh.at[is_v], add=True)

            return 0

        lax.fori_loop(0, NCH, chunk, 0)

        pltpu.sync_copy(den_v.at[pl.ds(0, N)], den_st.at[pl.ds(sid * N, N)])
        plsc.subcore_barrier()

        pltpu.sync_copy(u_sh.at[pl.ds(zoff, DPT)],
                        u_out.at[pl.ds(cid * N + zoff, DPT)])
        if DREM:
            @pl.when(sid == NS - 1)
            def _():
                pltpu.sync_copy(u_sh.at[pl.ds(NS * DPT, DREM)],
                                u_out.at[pl.ds(cid * N + NS * DPT, DREM)])
        _tile_reduce_dump(den_st, acc_v, tmp_v, ta_v, tt_v, den_out,
                          N, DPT, DREM, sid, cid)

    return k1


@functools.lru_cache(maxsize=None)
def _norm_kernel(N, D, softmax_eps):
    CH = 128
    FULLC = N // CH
    TREM = N - FULLC * CH
    TAILW = FULLC % NW
    ROUNDS = (FULLC + NW - 1) // NW
    DK = D // LANES
    assert TREM % 16 == 0

    @functools.partial(
        pl.kernel,
        out_type=jax.ShapeDtypeStruct((N, D), F32),
        mesh=_mesh(),
        scratch_types=[
            pltpu.VMEM((CH, D), F32),
            pltpu.VMEM((CH, D), F32),
            pltpu.VMEM((CH,), F32),
            pltpu.VMEM((CH,), F32),
        ],
    )
    def k(x_hbm, d_hbm, o_hbm, a_v, b_v, da_v, db_v):
        sid = lax.axis_index("s")
        cid = lax.axis_index("c")
        wid = sid * NC + cid
        iota = lax.iota(jnp.int32, LANES)
        onev = jnp.full((LANES,), 1.0, F32)
        epsv = jnp.full((LANES,), 1e-16, F32)

        def do(base, nrows):
            pltpu.sync_copy(x_hbm.at[pl.ds(base, nrows)], a_v.at[pl.ds(0, nrows)])
            pltpu.sync_copy(x_hbm.at[pl.ds(N + base, nrows)], b_v.at[pl.ds(0, nrows)])
            pltpu.sync_copy(d_hbm.at[pl.ds(base, nrows)], da_v.at[pl.ds(0, nrows)])
            pltpu.sync_copy(d_hbm.at[pl.ds(N + base, nrows)], db_v.at[pl.ds(0, nrows)])

            def group(g, _):
                d = da_v[pl.ds(g * 16, 16)] + db_v[pl.ds(g * 16, 16)]
                if softmax_eps:
                    inv16 = onev / (d + epsv)
                else:
                    inv16 = onev / jnp.maximum(d, onev)
                for l in range(LANES):
                    inv = inv16.at[iota * 0 + l].get(
                        mode="promise_in_bounds", unique_indices=False)
                    r = g * 16 + l
                    for j in range(DK):
                        sl = pl.ds(j * LANES, LANES)
                        a_v[r, sl] = (a_v[r, sl] + b_v[r, sl]) * inv
                return 0

            lax.fori_loop(0, nrows // 16, group, 0)
            pltpu.sync_copy(a_v.at[pl.ds(0, nrows)], o_hbm.at[pl.ds(base, nrows)])

        for t in range(ROUNDS):
            c = wid + NW * t
            if (t + 1) * NW <= FULLC:
                do(c * CH, CH)
            else:
                @pl.when(c < FULLC)
                def _():
                    do(c * CH, CH)
        if TREM:
            @pl.when(wid == TAILW)
            def _():
                do(FULLC * CH, TREM)

    return k


@functools.lru_cache(maxsize=None)
def _cell_gather_kernel(NCELL, N, D):
    CPW = (NCELL // NW) // 8 * 8
    REM = NCELL - CPW * NW
    CK = _chunk_1d(CPW)
    NCH = CPW // CK
    DK = D // LANES

    scratch = [
        pltpu.VMEM((CK, D), F32),
        pltpu.VMEM((CK, D), F32),
        pltpu.VMEM((CK, D), F32),
        pltpu.VMEM((CK, D), F32),
        pltpu.VMEM((CK,), jnp.int32),
        pltpu.VMEM((CK,), jnp.int32),
        pltpu.VMEM((CK,), jnp.int32),
        pltpu.SemaphoreType.DMA,
        pltpu.SemaphoreType.DMA,
        pltpu.SemaphoreType.DMA,
    ]
    if REM:
        scratch += [
            pltpu.VMEM((REM,), jnp.int32),
            pltpu.VMEM((REM,), jnp.int32),
            pltpu.VMEM((REM,), jnp.int32),
        ]

    @functools.partial(
        pl.kernel,
        out_type=jax.ShapeDtypeStruct((NCELL, D), F32),
        mesh=_mesh(),
        scratch_types=scratch,
    )
    def k(f0_hbm, f1_hbm, f2_hbm, nagg_hbm, o_hbm,
          g0_v, g1_v, g2_v, o_v, i0_v, i1_v, i2_v, s0, s1, s2, *tails):
        sid = lax.axis_index("s")
        cid = lax.axis_index("c")
        wid = sid * NC + cid
        third = jnp.full((LANES,), 1.0 / 3.0, F32)

        def do(b, n, i0, i1, i2):
            pltpu.sync_copy(f0_hbm.at[pl.ds(b, n)], i0)
            pltpu.sync_copy(f1_hbm.at[pl.ds(b, n)], i1)
            pltpu.sync_copy(f2_hbm.at[pl.ds(b, n)], i2)
            c0 = pltpu.async_copy(nagg_hbm.at[i0], g0_v.at[pl.ds(0, n)], s0)
            c1 = pltpu.async_copy(nagg_hbm.at[i1], g1_v.at[pl.ds(0, n)], s1)
            c2 = pltpu.async_copy(nagg_hbm.at[i2], g2_v.at[pl.ds(0, n)], s2)
            c0.wait()
            c1.wait()
            c2.wait()

            def cell(i, _):
                for j in range(DK):
                    sl = pl.ds(j * LANES, LANES)
                    o_v[i, sl] = (g0_v[i, sl] + g1_v[i, sl] + g2_v[i, sl]) * third
                return 0

            lax.fori_loop(0, n, cell, 0)
            pltpu.sync_copy(o_v.at[pl.ds(0, n)], o_hbm.at[pl.ds(b, n)])

        def chunk(c, _):
            do(wid * CPW + c * CK, CK, i0_v, i1_v, i2_v)
            return 0

        lax.fori_loop(0, NCH, chunk, 0)
        if REM:
            t0, t1, t2 = tails

            @pl.when(wid == NW - 1)
            def _():
                do(NW * CPW, REM, t0, t1, t2)

    return k


@functools.lru_cache(maxsize=None)
def _cell_scatter_kernel(NCELL, N, D):
    CPW = (NCELL // NW) // 8 * 8
    REM = NCELL - CPW * NW
    CK = _chunk_1d(CPW)
    NCH = CPW // CK
    DPT = (N // NS) // 8 * 8
    DREM = N - DPT * NS

    scratch = [
        pltpu.VMEM((CK, D), F32),
        pltpu.VMEM((CK,), jnp.int32),
        pltpu.VMEM((CK,), jnp.int32),
        pltpu.VMEM((CK,), jnp.int32),
        pltpu.VMEM((CK + 16,), jnp.int32),
        pltpu.VMEM((CK + 16,), jnp.int32),
        pltpu.VMEM((CK + 16,), jnp.int32),
        pltpu.VMEM((N + 16,), F32),
        pltpu.VMEM((DPT,), F32),
        pltpu.VMEM((DPT,), F32),
        pltpu.VMEM((16,), F32),
        pltpu.VMEM((16,), F32),
        pltpu.VMEM_SHARED((N, D), F32),
        pltpu.VMEM_SHARED((NS * N,), F32),
    ]
    if REM:
        scratch += [
            pltpu.VMEM((REM,), jnp.int32),
            pltpu.VMEM((REM,), jnp.int32),
            pltpu.VMEM((REM,), jnp.int32),
            pltpu.VMEM((REM + 16,), jnp.int32),
            pltpu.VMEM((REM + 16,), jnp.int32),
            pltpu.VMEM((REM + 16,), jnp.int32),
        ]

    @functools.partial(
        pl.kernel,
        out_type=(
            jax.ShapeDtypeStruct((NC * N, D), F32),
            jax.ShapeDtypeStruct((NC * N,), F32),
        ),
        mesh=_mesh(),
        scratch_types=scratch,
    )
    def k(cell_hbm, f0_hbm, f1_hbm, f2_hbm, zn_hbm, s_out, c_out,
          buf_v, i0_v, i1_v, i2_v, i0p_v, i1p_v, i2p_v, cnt_v,
          acc_v, tmp_v, ta_v, tt_v, s_sh, cnt_st, *tails):
        sid = lax.axis_index("s")
        cid = lax.axis_index("c")
        wid = sid * NC + cid
        iota = lax.iota(jnp.int32, LANES)
        zeros16 = jnp.zeros((LANES,), F32)
        onev = jnp.full((LANES,), 1.0, F32)
        eye0 = jnp.where(iota == 0, onev, zeros16)

        def zc(i, _):
            cnt_v[pl.ds(i * 16, 16)] = zeros16
            return 0

        lax.fori_loop(0, (N + 16) // 16, zc, 0)
        zoff = sid * DPT
        pltpu.sync_copy(zn_hbm.at[pl.ds(zoff, DPT)], s_sh.at[pl.ds(zoff, DPT)])
        if DREM:
            @pl.when(sid == NS - 1)
            def _():
                pltpu.sync_copy(zn_hbm.at[pl.ds(NS * DPT, DREM)],
                                s_sh.at[pl.ds(NS * DPT, DREM)])
        plsc.subcore_barrier()

        def do(b, n, i0, i1, i2, i0p, i1p, i2p):
            pltpu.sync_copy(cell_hbm.at[pl.ds(b, n)], buf_v.at[pl.ds(0, n)])
            pltpu.sync_copy(f0_hbm.at[pl.ds(b, n)], i0)
            pltpu.sync_copy(f1_hbm.at[pl.ds(b, n)], i1)
            pltpu.sync_copy(f2_hbm.at[pl.ds(b, n)], i2)
            pltpu.sync_copy(f0_hbm.at[pl.ds(b, n)], i0p.at[pl.ds(0, n)])
            pltpu.sync_copy(f1_hbm.at[pl.ds(b, n)], i1p.at[pl.ds(0, n)])
            pltpu.sync_copy(f2_hbm.at[pl.ds(b, n)], i2p.at[pl.ds(0, n)])
            pltpu.sync_copy(buf_v.at[pl.ds(0, n)], s_sh.at[i0], add=True)
            pltpu.sync_copy(buf_v.at[pl.ds(0, n)], s_sh.at[i1], add=True)
            pltpu.sync_copy(buf_v.at[pl.ds(0, n)], s_sh.at[i2], add=True)

            def cellcnt(i, _):
                for ipref in (i0p, i1p, i2p):
                    nn = ipref[pl.ds(i, 16)][0]
                    cnt_v[pl.ds(nn, 16)] = cnt_v[pl.ds(nn, 16)] + eye0
                return 0

            lax.fori_loop(0, n, cellcnt, 0)

        def chunk(c, _):
            do(wid * CPW + c * CK, CK, i0_v, i1_v, i2_v, i0p_v, i1p_v, i2p_v)
            return 0

        lax.fori_loop(0, NCH, chunk, 0)
        if REM:
            t0, t1, t2, t0p, t1p, t2p = tails

            @pl.when(wid == NW - 1)
            def _():
                do(NW * CPW, REM, t0, t1, t2, t0p, t1p, t2p)

        pltpu.sync_copy(cnt_v.at[pl.ds(0, N)], cnt_st.at[pl.ds(sid * N, N)])
        plsc.subcore_barrier()
        pltpu.sync_copy(s_sh.at[pl.ds(zoff, DPT)],
                        s_out.at[pl.ds(cid * N + zoff, DPT)])
        if DREM:
            @pl.when(sid == NS - 1)
            def _():
                pltpu.sync_copy(s_sh.at[pl.ds(NS * DPT, DREM)],
                                s_out.at[pl.ds(cid * N + NS * DPT, DREM)])
        _tile_reduce_dump(cnt_st, acc_v, tmp_v, ta_v, tt_v, c_out,
                          N, DPT, DREM, sid, cid)

    return k


def _matmul(cell_attr, cell_agg, w1, w2, b8):
    M, D = cell_attr.shape
    BM = 2000
    assert M % BM == 0

    def body(a_ref, g_ref, w1_ref, w2_ref, b_ref, o_ref):
        o_ref[...] = (
            jnp.dot(a_ref[...], w1_ref[...], preferred_element_type=F32)
            + jnp.dot(g_ref[...], w2_ref[...], preferred_element_type=F32)
            + b_ref[0:1, :]
        )

    return pl.pallas_call(
        body,
        grid=(M // BM,),
        in_specs=[
            pl.BlockSpec((BM, D), lambda i: (i, 0)),
            pl.BlockSpec((BM, D), lambda i: (i, 0)),
            pl.BlockSpec((D, D), lambda i: (0, 0)),
            pl.BlockSpec((D, D), lambda i: (0, 0)),
            pl.BlockSpec((8, D), lambda i: (0, 0)),
        ],
        out_specs=pl.BlockSpec((BM, D), lambda i: (i, 0)),
        out_shape=jax.ShapeDtypeStruct((M, D), F32),
    )(cell_attr, cell_agg, w1, w2, b8)


def kernel(cell_attr, edge_attr, node_embedding, edge_index, face, W, b):
    E, D = edge_attr.shape
    N = node_embedding.shape[0]
    NCELL = cell_attr.shape[0]

    senders = edge_index[0]
    receivers = edge_index[1]
    f0, f1, f2 = face[0], face[1], face[2]

    zn = jnp.zeros((N, D), F32)
    u_parts, den_parts = _edge_kernel(E, N, D)(
        edge_attr, senders, receivers, node_embedding, zn)
    node_agg = _norm_kernel(N, D, True)(u_parts, den_parts)
    cell_agg = _cell_gather_kernel(NCELL, N, D)(f0, f1, f2, node_agg)
    b8 = jnp.broadcast_to(b.reshape(1, D), (8, D))
    cell_new = _matmul(cell_attr, cell_agg, W[:D], W[D:], b8)
    s_parts, c_parts = _cell_scatter_kernel(NCELL, N, D)(
        cell_new, f0, f1, f2, zn)
    node_attr = _norm_kernel(N, D, False)(s_parts, c_parts)
    return cell_new, node_attr


# speedup vs baseline: 15.4572x; 1.4697x over previous
import functools

import jax
import jax.numpy as jnp
from jax import lax
from jax.experimental import pallas as pl
from jax.experimental.pallas import tpu as pltpu
from jax.experimental.pallas import tpu_sc as plsc

NC = 2
NS = 16
NW = NC * NS
LANES = 16

F32 = jnp.float32

def _mesh():
    return plsc.VectorSubcoreMesh(
        core_axis_name="c", subcore_axis_name="s", num_cores=NC, num_subcores=NS
    )


def _chunk_1d(n, cap=128):
    for k in range(cap, 0, -8):
        if n % k == 0:
            return k
    raise ValueError(f"no 8-aligned chunk for {n}")


def _hsum(x):
    iota = lax.iota(jnp.int32, LANES)
    for m in (1, 2, 4, 8):
        x = x + x.at[iota ^ m].get(mode="promise_in_bounds", unique_indices=True)
    return x


def _windows(n):
    wins = [(s, 0) for s in range(0, n - 15, 16)]
    cov = 16 * len(wins)
    if cov < n:
        wins.append((n - 16, 16 - (n - cov)))
    return wins


def _tile_reduce_dump(src_sh, acc_v, tmp_v, ta_v, tt_v, out_hbm, N, DPT, DREM,
                      sid, cid):
    zoff = sid * DPT
    pltpu.sync_copy(src_sh.at[pl.ds(zoff, DPT)], acc_v)
    for w in range(1, NS):
        pltpu.sync_copy(src_sh.at[pl.ds(w * N + zoff, DPT)], tmp_v)

        def addw(i, _):
            acc_v[pl.ds(i * 16, 16)] = (acc_v[pl.ds(i * 16, 16)]
                                        + tmp_v[pl.ds(i * 16, 16)])
            return 0

        lax.fori_loop(0, DPT // 16, addw, 0)
    pltpu.sync_copy(acc_v, out_hbm.at[pl.ds(cid * N + zoff, DPT)])
    if DREM:
        @pl.when(sid == NS - 1)
        def _():
            pltpu.sync_copy(src_sh.at[pl.ds(NS * DPT, DREM)], ta_v)
            for w in range(1, NS):
                pltpu.sync_copy(src_sh.at[pl.ds(w * N + NS * DPT, DREM)], tt_v)
                ta_v[...] = ta_v[...] + tt_v[...]
            pltpu.sync_copy(ta_v, out_hbm.at[pl.ds(cid * N + NS * DPT, DREM)])


@functools.lru_cache(maxsize=None)
def _edge_kernel(E, N, D):
    EPW = E // NW
    EK = _chunk_1d(EPW, cap=40)
    NCH = EPW // EK
    DPT = (N // NS) // 8 * 8
    DREM = N - DPT * NS
    DK = D // LANES
    inv_scale = 1.0 / float(D) ** 0.5
    NPAD = ((N + 127) // 128) * 128
    assert N % 16 == 0 and NCH % 2 == 0

    @functools.partial(
        pl.kernel,
        out_type=(
            jax.ShapeDtypeStruct((NC * N, D), F32),
            jax.ShapeDtypeStruct((NW * NPAD,), F32),
        ),
        mesh=_mesh(),
        scratch_types=[
            pltpu.VMEM((EK, D), F32), pltpu.VMEM((EK, D), F32),
            pltpu.VMEM((EK, D), F32), pltpu.VMEM((EK, D), F32),
            pltpu.VMEM((EK, D), F32), pltpu.VMEM((EK, D), F32),
            pltpu.VMEM((EK,), jnp.int32), pltpu.VMEM((EK,), jnp.int32),
            pltpu.VMEM((EK,), jnp.int32), pltpu.VMEM((EK,), jnp.int32),
            pltpu.VMEM((EK + 16,), jnp.int32), pltpu.VMEM((EK + 16,), jnp.int32),
            pltpu.VMEM((EK + 16,), jnp.int32), pltpu.VMEM((EK + 16,), jnp.int32),
            pltpu.VMEM((N + 16,), F32),
            pltpu.VMEM_SHARED((N, D), F32),
            pltpu.SemaphoreType.DMA, pltpu.SemaphoreType.DMA,
            pltpu.SemaphoreType.DMA, pltpu.SemaphoreType.DMA,
            pltpu.SemaphoreType.DMA, pltpu.SemaphoreType.DMA,
        ],
    )
    def k1(edge_hbm, snd_hbm, rcv_hbm, nemb_hbm, zn_hbm, u_out, den_out,
           e_v0, e_v1, nr_v0, nr_v1, ns_v0, ns_v1, ir_v0, ir_v1,
           is_v0, is_v1, irp_v0, irp_v1, isp_v0, isp_v1, den_v, u_sh,
           sl0, sl1, sgr0, sgr1, sgs0, sgs1):
        sid = lax.axis_index("s")
        cid = lax.axis_index("c")
        wid = sid * NC + cid
        iota = lax.iota(jnp.int32, LANES)
        zeros16 = jnp.zeros((LANES,), F32)
        onev = jnp.full((LANES,), 1.0, F32)
        eye0 = jnp.where(iota == 0, onev, zeros16)
        inv_scale_v = jnp.full((LANES,), inv_scale, F32)
        ebase = wid * EPW

        bufs = (
            (e_v0, nr_v0, ns_v0, ir_v0, is_v0, irp_v0, isp_v0, sl0, sgr0, sgs0),
            (e_v1, nr_v1, ns_v1, ir_v1, is_v1, irp_v1, isp_v1, sl1, sgr1, sgs1),
        )

        def zden(i, _):
            den_v[pl.ds(i * 16, 16)] = zeros16
            return 0

        lax.fori_loop(0, (N + 16) // 16, zden, 0)
        zoff = sid * DPT
        pltpu.sync_copy(zn_hbm.at[pl.ds(zoff, DPT)], u_sh.at[pl.ds(zoff, DPT)])
        if DREM:
            @pl.when(sid == NS - 1)
            def _():
                pltpu.sync_copy(zn_hbm.at[pl.ds(NS * DPT, DREM)],
                                u_sh.at[pl.ds(NS * DPT, DREM)])
        plsc.subcore_barrier()

        def load_list(c, B):
            e_v, nr_v, ns_v, ir_v, is_v, irp_v, isp_v = B[:7]
            b = ebase + c * EK
            return [
                (snd_hbm.at[pl.ds(b, EK)], is_v),
                (rcv_hbm.at[pl.ds(b, EK)], ir_v),
                (snd_hbm.at[pl.ds(b, EK)], isp_v.at[pl.ds(0, EK)]),
                (rcv_hbm.at[pl.ds(b, EK)], irp_v.at[pl.ds(0, EK)]),
                (edge_hbm.at[pl.ds(b, EK)], e_v),
            ]

        def issue_loads(c, B):
            for s, d in load_list(c, B):
                pltpu.async_copy(s, d, B[7])

        def wait_loads(c, B):
            for s, d in load_list(c, B):
                pltpu.make_async_copy(s, d, B[7]).wait()

        def issue_gathers(B):
            pltpu.async_copy(nemb_hbm.at[B[3]], B[1], B[8])
            pltpu.async_copy(nemb_hbm.at[B[4]], B[2], B[9])

        def wait_gathers(B):
            pltpu.make_async_copy(nemb_hbm.at[B[3]], B[1], B[8]).wait()
            pltpu.make_async_copy(nemb_hbm.at[B[4]], B[2], B[9]).wait()

        def compute_scatter(B):
            e_v, nr_v, ns_v, ir_v, is_v, irp_v, isp_v = B[:7]

            def edge(i, _):
                es = [e_v[i, pl.ds(j * LANES, LANES)] for j in range(DK)]
                accr = es[0] * nr_v[i, pl.ds(0, LANES)]
                accs = es[0] * ns_v[i, pl.ds(0, LANES)]
                for j in range(1, DK):
                    accr = accr + es[j] * nr_v[i, pl.ds(j * LANES, LANES)]
                    accs = accs + es[j] * ns_v[i, pl.ds(j * LANES, LANES)]
                exr = jnp.exp(_hsum(accr) * inv_scale_v)
                exs = jnp.exp(_hsum(accs) * inv_scale_v)
                for j in range(DK):
                    nr_v[i, pl.ds(j * LANES, LANES)] = es[j] * exr
                    ns_v[i, pl.ds(j * LANES, LANES)] = es[j] * exs
                nri = irp_v[pl.ds(i, 16)][0]
                den_v[pl.ds(nri, 16)] = den_v[pl.ds(nri, 16)] + exr * eye0
                nsi = isp_v[pl.ds(i, 16)][0]
                den_v[pl.ds(nsi, 16)] = den_v[pl.ds(nsi, 16)] + exs * eye0
                return 0

            lax.fori_loop(0, EK, edge, 0)
            pltpu.sync_copy(nr_v, u_sh.at[ir_v], add=True)
            pltpu.sync_copy(ns_v, u_sh.at[is_v], add=True)

        issue_loads(0, bufs[0])
        wait_loads(0, bufs[0])
        issue_gathers(bufs[0])
        if NCH > 1:
            issue_loads(1, bufs[1])

        def pair(p, _):
            for bpar in (0, 1):
                c = 2 * p + bpar
                B = bufs[bpar]
                Bn = bufs[1 - bpar]

                @pl.when(c + 1 < NCH)
                def _():
                    wait_loads(c + 1, Bn)
                    issue_gathers(Bn)

                wait_gathers(B)
                compute_scatter(B)

                @pl.when(c + 2 < NCH)
                def _():
                    issue_loads(c + 2, B)
            return 0

        lax.fori_loop(0, NCH // 2, pair, 0)
        plsc.subcore_barrier()

        pltpu.sync_copy(u_sh.at[pl.ds(zoff, DPT)],
                        u_out.at[pl.ds(cid * N + zoff, DPT)])
        if DREM:
            @pl.when(sid == NS - 1)
            def _():
                pltpu.sync_copy(u_sh.at[pl.ds(NS * DPT, DREM)],
                                u_out.at[pl.ds(cid * N + NS * DPT, DREM)])
        pltpu.sync_copy(den_v.at[pl.ds(0, N)],
                        den_out.at[pl.ds(wid * NPAD, N)])

    return k1


@functools.lru_cache(maxsize=None)
def _norm_kernel(N, D, softmax_eps, many_parts):
    CH = 128
    FULLC = N // CH
    TREM = N - FULLC * CH
    TAILW = FULLC % NW
    ROUNDS = (FULLC + NW - 1) // NW
    DK = D // LANES
    NPAD = ((N + 127) // 128) * 128
    assert TREM % 16 == 0

    scratch = [
        pltpu.VMEM((CH, D), F32),
        pltpu.VMEM((CH, D), F32),
    ]
    if many_parts:
        scratch.append(pltpu.VMEM((NW, CH), F32))
    else:
        scratch += [pltpu.VMEM((CH,), F32), pltpu.VMEM((CH,), F32)]

    @functools.partial(
        pl.kernel,
        out_type=jax.ShapeDtypeStruct((N, D), F32),
        mesh=_mesh(),
        scratch_types=scratch,
    )
    def k(x_hbm, d_hbm, o_hbm, a_v, b_v, *dbufs):
        sid = lax.axis_index("s")
        cid = lax.axis_index("c")
        wid = sid * NC + cid
        iota = lax.iota(jnp.int32, LANES)
        onev = jnp.full((LANES,), 1.0, F32)
        epsv = jnp.full((LANES,), 1e-16, F32)

        def do(base, nrows):
            pltpu.sync_copy(x_hbm.at[pl.ds(base, nrows)], a_v.at[pl.ds(0, nrows)])
            pltpu.sync_copy(x_hbm.at[pl.ds(N + base, nrows)], b_v.at[pl.ds(0, nrows)])
            if many_parts:
                dbuf = dbufs[0]
                pltpu.sync_copy(d_hbm.at[:, pl.ds(base, CH)], dbuf)
            else:
                da_v, db_v = dbufs
                pltpu.sync_copy(d_hbm.at[pl.ds(base, nrows)], da_v.at[pl.ds(0, nrows)])
                pltpu.sync_copy(d_hbm.at[pl.ds(N + base, nrows)], db_v.at[pl.ds(0, nrows)])

            def group(g, _):
                if many_parts:
                    dbuf = dbufs[0]
                    d = dbuf[0, pl.ds(g * 16, 16)]
                    for w in range(1, NW):
                        d = d + dbuf[w, pl.ds(g * 16, 16)]
                else:
                    da_v, db_v = dbufs
                    d = da_v[pl.ds(g * 16, 16)] + db_v[pl.ds(g * 16, 16)]
                if softmax_eps:
                    inv16 = onev / (d + epsv)
                else:
                    inv16 = onev / jnp.maximum(d, onev)
                for l in range(LANES):
                    inv = inv16.at[iota * 0 + l].get(
                        mode="promise_in_bounds", unique_indices=False)
                    r = g * 16 + l
                    for j in range(DK):
                        sl = pl.ds(j * LANES, LANES)
                        a_v[r, sl] = (a_v[r, sl] + b_v[r, sl]) * inv
                return 0

            lax.fori_loop(0, nrows // 16, group, 0)
            pltpu.sync_copy(a_v.at[pl.ds(0, nrows)], o_hbm.at[pl.ds(base, nrows)])

        for t in range(ROUNDS):
            c = wid + NW * t
            if (t + 1) * NW <= FULLC:
                do(c * CH, CH)
            else:
                @pl.when(c < FULLC)
                def _():
                    do(c * CH, CH)
        if TREM:
            @pl.when(wid == TAILW)
            def _():
                do(FULLC * CH, TREM)

    return k


@functools.lru_cache(maxsize=None)
def _cell_gather_kernel(NCELL, N, D):
    CPW = (NCELL // NW) // 8 * 8
    REM = NCELL - CPW * NW
    CK = _chunk_1d(CPW)
    NCH = CPW // CK
    DK = D // LANES

    scratch = [
        pltpu.VMEM((CK, D), F32),
        pltpu.VMEM((CK, D), F32),
        pltpu.VMEM((CK, D), F32),
        pltpu.VMEM((CK, D), F32),
        pltpu.VMEM((CK,), jnp.int32),
        pltpu.VMEM((CK,), jnp.int32),
        pltpu.VMEM((CK,), jnp.int32),
        pltpu.SemaphoreType.DMA,
        pltpu.SemaphoreType.DMA,
        pltpu.SemaphoreType.DMA,
    ]
    if REM:
        scratch += [
            pltpu.VMEM((REM,), jnp.int32),
            pltpu.VMEM((REM,), jnp.int32),
            pltpu.VMEM((REM,), jnp.int32),
        ]

    @functools.partial(
        pl.kernel,
        out_type=jax.ShapeDtypeStruct((NCELL, D), F32),
        mesh=_mesh(),
        scratch_types=scratch,
    )
    def k(f0_hbm, f1_hbm, f2_hbm, nagg_hbm, o_hbm,
          g0_v, g1_v, g2_v, o_v, i0_v, i1_v, i2_v, s0, s1, s2, *tails):
        sid = lax.axis_index("s")
        cid = lax.axis_index("c")
        wid = sid * NC + cid
        third = jnp.full((LANES,), 1.0 / 3.0, F32)

        def do(b, n, i0, i1, i2):
            pltpu.sync_copy(f0_hbm.at[pl.ds(b, n)], i0)
            pltpu.sync_copy(f1_hbm.at[pl.ds(b, n)], i1)
            pltpu.sync_copy(f2_hbm.at[pl.ds(b, n)], i2)
            c0 = pltpu.async_copy(nagg_hbm.at[i0], g0_v.at[pl.ds(0, n)], s0)
            c1 = pltpu.async_copy(nagg_hbm.at[i1], g1_v.at[pl.ds(0, n)], s1)
            c2 = pltpu.async_copy(nagg_hbm.at[i2], g2_v.at[pl.ds(0, n)], s2)
            c0.wait()
            c1.wait()
            c2.wait()

            def cell(i, _):
                for j in range(DK):
                    sl = pl.ds(j * LANES, LANES)
                    o_v[i, sl] = (g0_v[i, sl] + g1_v[i, sl] + g2_v[i, sl]) * third
                return 0

            lax.fori_loop(0, n, cell, 0)
            pltpu.sync_copy(o_v.at[pl.ds(0, n)], o_hbm.at[pl.ds(b, n)])

        def chunk(c, _):
            do(wid * CPW + c * CK, CK, i0_v, i1_v, i2_v)
            return 0

        lax.fori_loop(0, NCH, chunk, 0)
        if REM:
            t0, t1, t2 = tails

            @pl.when(wid == NW - 1)
            def _():
                do(NW * CPW, REM, t0, t1, t2)

    return k


@functools.lru_cache(maxsize=None)
def _cell_scatter_kernel(NCELL, N, D):
    CPW = (NCELL // NW) // 8 * 8
    REM = NCELL - CPW * NW
    CK = _chunk_1d(CPW)
    NCH = CPW // CK
    DPT = (N // NS) // 8 * 8
    DREM = N - DPT * NS

    scratch = [
        pltpu.VMEM((CK, D), F32),
        pltpu.VMEM((CK,), jnp.int32),
        pltpu.VMEM((CK,), jnp.int32),
        pltpu.VMEM((CK,), jnp.int32),
        pltpu.VMEM((CK + 16,), jnp.int32),
        pltpu.VMEM((CK + 16,), jnp.int32),
        pltpu.VMEM((CK + 16,), jnp.int32),
        pltpu.VMEM((N + 16,), F32),
        pltpu.VMEM((DPT,), F32),
        pltpu.VMEM((DPT,), F32),
        pltpu.VMEM((16,), F32),
        pltpu.VMEM((16,), F32),
        pltpu.VMEM_SHARED((N, D), F32),
        pltpu.VMEM_SHARED((NS * N,), F32),
    ]
    if REM:
        scratch += [
            pltpu.VMEM((REM,), jnp.int32),
            pltpu.VMEM((REM,), jnp.int32),
            pltpu.VMEM((REM,), jnp.int32),
            pltpu.VMEM((REM + 16,), jnp.int32),
            pltpu.VMEM((REM + 16,), jnp.int32),
            pltpu.VMEM((REM + 16,), jnp.int32),
        ]

    @functools.partial(
        pl.kernel,
        out_type=(
            jax.ShapeDtypeStruct((NC * N, D), F32),
            jax.ShapeDtypeStruct((NC * N,), F32),
        ),
        mesh=_mesh(),
        scratch_types=scratch,
    )
    def k(cell_hbm, f0_hbm, f1_hbm, f2_hbm, zn_hbm, s_out, c_out,
          buf_v, i0_v, i1_v, i2_v, i0p_v, i1p_v, i2p_v, cnt_v,
          acc_v, tmp_v, ta_v, tt_v, s_sh, cnt_st, *tails):
        sid = lax.axis_index("s")
        cid = lax.axis_index("c")
        wid = sid * NC + cid
        iota = lax.iota(jnp.int32, LANES)
        zeros16 = jnp.zeros((LANES,), F32)
        onev = jnp.full((LANES,), 1.0, F32)
        eye0 = jnp.where(iota == 0, onev, zeros16)

        def zc(i, _):
            cnt_v[pl.ds(i * 16, 16)] = zeros16
            return 0

        lax.fori_loop(0, (N + 16) // 16, zc, 0)
        zoff = sid * DPT
        pltpu.sync_copy(zn_hbm.at[pl.ds(zoff, DPT)], s_sh.at[pl.ds(zoff, DPT)])
        if DREM:
            @pl.when(sid == NS - 1)
            def _():
                pltpu.sync_copy(zn_hbm.at[pl.ds(NS * DPT, DREM)],
                                s_sh.at[pl.ds(NS * DPT, DREM)])
        plsc.subcore_barrier()

        def do(b, n, i0, i1, i2, i0p, i1p, i2p):
            pltpu.sync_copy(cell_hbm.at[pl.ds(b, n)], buf_v.at[pl.ds(0, n)])
            pltpu.sync_copy(f0_hbm.at[pl.ds(b, n)], i0)
            pltpu.sync_copy(f1_hbm.at[pl.ds(b, n)], i1)
            pltpu.sync_copy(f2_hbm.at[pl.ds(b, n)], i2)
            pltpu.sync_copy(f0_hbm.at[pl.ds(b, n)], i0p.at[pl.ds(0, n)])
            pltpu.sync_copy(f1_hbm.at[pl.ds(b, n)], i1p.at[pl.ds(0, n)])
            pltpu.sync_copy(f2_hbm.at[pl.ds(b, n)], i2p.at[pl.ds(0, n)])
            pltpu.sync_copy(buf_v.at[pl.ds(0, n)], s_sh.at[i0], add=True)
            pltpu.sync_copy(buf_v.at[pl.ds(0, n)], s_sh.at[i1], add=True)
            pltpu.sync_copy(buf_v.at[pl.ds(0, n)], s_sh.at[i2], add=True)

            def cellcnt(i, _):
                for ipref in (i0p, i1p, i2p):
                    nn = ipref[pl.ds(i, 16)][0]
                    cnt_v[pl.ds(nn, 16)] = cnt_v[pl.ds(nn, 16)] + eye0
                return 0

            lax.fori_loop(0, n, cellcnt, 0)

        def chunk(c, _):
            do(wid * CPW + c * CK, CK, i0_v, i1_v, i2_v, i0p_v, i1p_v, i2p_v)
            return 0

        lax.fori_loop(0, NCH, chunk, 0)
        if REM:
            t0, t1, t2, t0p, t1p, t2p = tails

            @pl.when(wid == NW - 1)
            def _():
                do(NW * CPW, REM, t0, t1, t2, t0p, t1p, t2p)

        pltpu.sync_copy(cnt_v.at[pl.ds(0, N)], cnt_st.at[pl.ds(sid * N, N)])
        plsc.subcore_barrier()
        pltpu.sync_copy(s_sh.at[pl.ds(zoff, DPT)],
                        s_out.at[pl.ds(cid * N + zoff, DPT)])
        if DREM:
            @pl.when(sid == NS - 1)
            def _():
                pltpu.sync_copy(s_sh.at[pl.ds(NS * DPT, DREM)],
                                s_out.at[pl.ds(cid * N + NS * DPT, DREM)])
        _tile_reduce_dump(cnt_st, acc_v, tmp_v, ta_v, tt_v, c_out,
                          N, DPT, DREM, sid, cid)

    return k


def _matmul(cell_attr, cell_agg, w1, w2, b8):
    M, D = cell_attr.shape
    BM = 2000
    assert M % BM == 0

    def body(a_ref, g_ref, w1_ref, w2_ref, b_ref, o_ref):
        o_ref[...] = (
            jnp.dot(a_ref[...], w1_ref[...], preferred_element_type=F32)
            + jnp.dot(g_ref[...], w2_ref[...], preferred_element_type=F32)
            + b_ref[0:1, :]
        )

    return pl.pallas_call(
        body,
        grid=(M // BM,),
        in_specs=[
            pl.BlockSpec((BM, D), lambda i: (i, 0)),
            pl.BlockSpec((BM, D), lambda i: (i, 0)),
            pl.BlockSpec((D, D), lambda i: (0, 0)),
            pl.BlockSpec((D, D), lambda i: (0, 0)),
            pl.BlockSpec((8, D), lambda i: (0, 0)),
        ],
        out_specs=pl.BlockSpec((BM, D), lambda i: (i, 0)),
        out_shape=jax.ShapeDtypeStruct((M, D), F32),
    )(cell_attr, cell_agg, w1, w2, b8)


def kernel(cell_attr, edge_attr, node_embedding, edge_index, face, W, b):
    E, D = edge_attr.shape
    N = node_embedding.shape[0]
    NCELL = cell_attr.shape[0]

    senders = edge_index[0]
    receivers = edge_index[1]
    f0, f1, f2 = face[0], face[1], face[2]

    zn = jnp.zeros((N, D), F32)
    u_parts, den_parts = _edge_kernel(E, N, D)(
        edge_attr, senders, receivers, node_embedding, zn)
    npad = ((N + 127) // 128) * 128
    node_agg = _norm_kernel(N, D, True, True)(
        u_parts, den_parts.reshape(NW, npad))
    cell_agg = _cell_gather_kernel(NCELL, N, D)(f0, f1, f2, node_agg)
    b8 = jnp.broadcast_to(b.reshape(1, D), (8, D))
    cell_new = _matmul(cell_attr, cell_agg, W[:D], W[D:], b8)
    s_parts, c_parts = _cell_scatter_kernel(NCELL, N, D)(
        cell_new, f0, f1, f2, zn)
    node_attr = _norm_kernel(N, D, False, False)(s_parts, c_parts)
    return cell_new, node_attr


# speedup vs baseline: 17.4220x; 1.1271x over previous
import functools

import jax
import jax.numpy as jnp
from jax import lax
from jax.experimental import pallas as pl
from jax.experimental.pallas import tpu as pltpu
from jax.experimental.pallas import tpu_sc as plsc

NC = 2
NS = 16
NW = NC * NS
LANES = 16

F32 = jnp.float32

def _mesh():
    return plsc.VectorSubcoreMesh(
        core_axis_name="c", subcore_axis_name="s", num_cores=NC, num_subcores=NS
    )


def _chunk_1d(n, cap=128):
    for k in range(cap, 0, -8):
        if n % k == 0:
            return k
    raise ValueError(f"no 8-aligned chunk for {n}")


def _hsum(x):
    iota = lax.iota(jnp.int32, LANES)
    for m in (1, 2, 4, 8):
        x = x + x.at[iota ^ m].get(mode="promise_in_bounds", unique_indices=True)
    return x


def _windows(n):
    wins = [(s, 0) for s in range(0, n - 15, 16)]
    cov = 16 * len(wins)
    if cov < n:
        wins.append((n - 16, 16 - (n - cov)))
    return wins


def _tile_reduce_dump(src_sh, acc_v, tmp_v, ta_v, tt_v, out_hbm, N, DPT, DREM,
                      sid, cid):
    zoff = sid * DPT
    pltpu.sync_copy(src_sh.at[pl.ds(zoff, DPT)], acc_v)
    for w in range(1, NS):
        pltpu.sync_copy(src_sh.at[pl.ds(w * N + zoff, DPT)], tmp_v)

        def addw(i, _):
            acc_v[pl.ds(i * 16, 16)] = (acc_v[pl.ds(i * 16, 16)]
                                        + tmp_v[pl.ds(i * 16, 16)])
            return 0

        lax.fori_loop(0, DPT // 16, addw, 0)
    pltpu.sync_copy(acc_v, out_hbm.at[pl.ds(cid * N + zoff, DPT)])
    if DREM:
        @pl.when(sid == NS - 1)
        def _():
            pltpu.sync_copy(src_sh.at[pl.ds(NS * DPT, DREM)], ta_v)
            for w in range(1, NS):
                pltpu.sync_copy(src_sh.at[pl.ds(w * N + NS * DPT, DREM)], tt_v)
                ta_v[...] = ta_v[...] + tt_v[...]
            pltpu.sync_copy(ta_v, out_hbm.at[pl.ds(cid * N + NS * DPT, DREM)])


@functools.lru_cache(maxsize=None)
def _edge_kernel(E, N, D):
    EPW = E // NW
    EK = _chunk_1d(EPW, cap=40)
    NCH = EPW // EK
    DPT = (N // NS) // 8 * 8
    DREM = N - DPT * NS
    DK = D // LANES
    inv_scale = 1.0 / float(D) ** 0.5
    NPAD = ((N + 127) // 128) * 128
    assert N % 16 == 0 and NCH % 2 == 0 and EK % 2 == 0

    @functools.partial(
        pl.kernel,
        out_type=(
            jax.ShapeDtypeStruct((NC * N, D), F32),
            jax.ShapeDtypeStruct((NW * NPAD,), F32),
        ),
        mesh=_mesh(),
        scratch_types=[
            pltpu.VMEM((EK, D), F32), pltpu.VMEM((EK, D), F32),
            pltpu.VMEM((EK, D), F32), pltpu.VMEM((EK, D), F32),
            pltpu.VMEM((EK, D), F32), pltpu.VMEM((EK, D), F32),
            pltpu.VMEM((EK,), jnp.int32), pltpu.VMEM((EK,), jnp.int32),
            pltpu.VMEM((EK,), jnp.int32), pltpu.VMEM((EK,), jnp.int32),
            pltpu.VMEM((EK + 16,), jnp.int32), pltpu.VMEM((EK + 16,), jnp.int32),
            pltpu.VMEM((EK + 16,), jnp.int32), pltpu.VMEM((EK + 16,), jnp.int32),
            pltpu.VMEM((N + 16,), F32),
            pltpu.VMEM_SHARED((N, D), F32),
            pltpu.SemaphoreType.DMA, pltpu.SemaphoreType.DMA,
            pltpu.SemaphoreType.DMA, pltpu.SemaphoreType.DMA,
            pltpu.SemaphoreType.DMA, pltpu.SemaphoreType.DMA,
        ],
    )
    def k1(edge_hbm, snd_hbm, rcv_hbm, nemb_hbm, zn_hbm, u_out, den_out,
           e_v0, e_v1, nr_v0, nr_v1, ns_v0, ns_v1, ir_v0, ir_v1,
           is_v0, is_v1, irp_v0, irp_v1, isp_v0, isp_v1, den_v, u_sh,
           sl0, sl1, sgr0, sgr1, sgs0, sgs1):
        sid = lax.axis_index("s")
        cid = lax.axis_index("c")
        wid = sid * NC + cid
        iota = lax.iota(jnp.int32, LANES)
        zeros16 = jnp.zeros((LANES,), F32)
        onev = jnp.full((LANES,), 1.0, F32)
        eye0 = jnp.where(iota == 0, onev, zeros16)
        inv_scale_v = jnp.full((LANES,), inv_scale, F32)
        ebase = wid * EPW

        bufs = (
            (e_v0, nr_v0, ns_v0, ir_v0, is_v0, irp_v0, isp_v0, sl0, sgr0, sgs0),
            (e_v1, nr_v1, ns_v1, ir_v1, is_v1, irp_v1, isp_v1, sl1, sgr1, sgs1),
        )

        def zden(i, _):
            den_v[pl.ds(i * 16, 16)] = zeros16
            return 0

        lax.fori_loop(0, (N + 16) // 16, zden, 0)
        zoff = sid * DPT
        pltpu.sync_copy(zn_hbm.at[pl.ds(zoff, DPT)], u_sh.at[pl.ds(zoff, DPT)])
        if DREM:
            @pl.when(sid == NS - 1)
            def _():
                pltpu.sync_copy(zn_hbm.at[pl.ds(NS * DPT, DREM)],
                                u_sh.at[pl.ds(NS * DPT, DREM)])
        plsc.subcore_barrier()

        def load_list(c, B):
            e_v, nr_v, ns_v, ir_v, is_v, irp_v, isp_v = B[:7]
            b = ebase + c * EK
            return [
                (snd_hbm.at[pl.ds(b, EK)], is_v),
                (rcv_hbm.at[pl.ds(b, EK)], ir_v),
                (snd_hbm.at[pl.ds(b, EK)], isp_v.at[pl.ds(0, EK)]),
                (rcv_hbm.at[pl.ds(b, EK)], irp_v.at[pl.ds(0, EK)]),
                (edge_hbm.at[pl.ds(b, EK)], e_v),
            ]

        def issue_loads(c, B):
            for s, d in load_list(c, B):
                pltpu.async_copy(s, d, B[7])

        def wait_loads(c, B):
            for s, d in load_list(c, B):
                pltpu.make_async_copy(s, d, B[7]).wait()

        def issue_gathers(B):
            pltpu.async_copy(nemb_hbm.at[B[3]], B[1], B[8])
            pltpu.async_copy(nemb_hbm.at[B[4]], B[2], B[9])

        def wait_gathers(B):
            pltpu.make_async_copy(nemb_hbm.at[B[3]], B[1], B[8]).wait()
            pltpu.make_async_copy(nemb_hbm.at[B[4]], B[2], B[9]).wait()

        def compute_scatter(B):
            e_v, nr_v, ns_v, ir_v, is_v, irp_v, isp_v = B[:7]

            def edge(i2, _):
                exs_l = []
                for u in range(2):
                    i = i2 * 2 + u
                    es = [e_v[i, pl.ds(j * LANES, LANES)] for j in range(DK)]
                    accr = es[0] * nr_v[i, pl.ds(0, LANES)]
                    accs = es[0] * ns_v[i, pl.ds(0, LANES)]
                    for j in range(1, DK):
                        accr = accr + es[j] * nr_v[i, pl.ds(j * LANES, LANES)]
                        accs = accs + es[j] * ns_v[i, pl.ds(j * LANES, LANES)]
                    exr = jnp.exp(_hsum(accr) * inv_scale_v)
                    exs = jnp.exp(_hsum(accs) * inv_scale_v)
                    for j in range(DK):
                        nr_v[i, pl.ds(j * LANES, LANES)] = es[j] * exr
                        ns_v[i, pl.ds(j * LANES, LANES)] = es[j] * exs
                    exs_l.append((i, exr, exs))
                for (i, exr, exs) in exs_l:
                    nri = irp_v[pl.ds(i, 16)][0]
                    den_v[pl.ds(nri, 16)] = den_v[pl.ds(nri, 16)] + exr * eye0
                    nsi = isp_v[pl.ds(i, 16)][0]
                    den_v[pl.ds(nsi, 16)] = den_v[pl.ds(nsi, 16)] + exs * eye0
                return 0

            lax.fori_loop(0, EK // 2, edge, 0)
            pltpu.sync_copy(nr_v, u_sh.at[ir_v], add=True)
            pltpu.sync_copy(ns_v, u_sh.at[is_v], add=True)

        issue_loads(0, bufs[0])
        wait_loads(0, bufs[0])
        issue_gathers(bufs[0])
        if NCH > 1:
            issue_loads(1, bufs[1])

        def pair(p, _):
            for bpar in (0, 1):
                c = 2 * p + bpar
                B = bufs[bpar]
                Bn = bufs[1 - bpar]

                @pl.when(c + 1 < NCH)
                def _():
                    wait_loads(c + 1, Bn)
                    issue_gathers(Bn)

                wait_gathers(B)
                compute_scatter(B)

                @pl.when(c + 2 < NCH)
                def _():
                    issue_loads(c + 2, B)
            return 0

        lax.fori_loop(0, NCH // 2, pair, 0)
        plsc.subcore_barrier()

        pltpu.sync_copy(u_sh.at[pl.ds(zoff, DPT)],
                        u_out.at[pl.ds(cid * N + zoff, DPT)])
        if DREM:
            @pl.when(sid == NS - 1)
            def _():
                pltpu.sync_copy(u_sh.at[pl.ds(NS * DPT, DREM)],
                                u_out.at[pl.ds(cid * N + NS * DPT, DREM)])
        pltpu.sync_copy(den_v.at[pl.ds(0, N)],
                        den_out.at[pl.ds(wid * NPAD, N)])

    return k1


@functools.lru_cache(maxsize=None)
def _norm_kernel(N, D, softmax_eps, many_parts):
    CH = 128
    FULLC = N // CH
    TREM = N - FULLC * CH
    TAILW = FULLC % NW
    ROUNDS = (FULLC + NW - 1) // NW
    DK = D // LANES
    NPAD = ((N + 127) // 128) * 128
    assert TREM % 16 == 0

    scratch = [
        pltpu.VMEM((CH, D), F32),
        pltpu.VMEM((CH, D), F32),
    ]
    if many_parts:
        scratch.append(pltpu.VMEM((NW, CH), F32))
    else:
        scratch += [pltpu.VMEM((CH,), F32), pltpu.VMEM((CH,), F32)]

    @functools.partial(
        pl.kernel,
        out_type=jax.ShapeDtypeStruct((N, D), F32),
        mesh=_mesh(),
        scratch_types=scratch,
    )
    def k(x_hbm, d_hbm, o_hbm, a_v, b_v, *dbufs):
        sid = lax.axis_index("s")
        cid = lax.axis_index("c")
        wid = sid * NC + cid
        iota = lax.iota(jnp.int32, LANES)
        onev = jnp.full((LANES,), 1.0, F32)
        epsv = jnp.full((LANES,), 1e-16, F32)

        def do(base, nrows):
            pltpu.sync_copy(x_hbm.at[pl.ds(base, nrows)], a_v.at[pl.ds(0, nrows)])
            pltpu.sync_copy(x_hbm.at[pl.ds(N + base, nrows)], b_v.at[pl.ds(0, nrows)])
            if many_parts:
                dbuf = dbufs[0]
                pltpu.sync_copy(d_hbm.at[:, pl.ds(base, CH)], dbuf)
            else:
                da_v, db_v = dbufs
                pltpu.sync_copy(d_hbm.at[pl.ds(base, nrows)], da_v.at[pl.ds(0, nrows)])
                pltpu.sync_copy(d_hbm.at[pl.ds(N + base, nrows)], db_v.at[pl.ds(0, nrows)])

            def group(g, _):
                if many_parts:
                    dbuf = dbufs[0]
                    d = dbuf[0, pl.ds(g * 16, 16)]
                    for w in range(1, NW):
                        d = d + dbuf[w, pl.ds(g * 16, 16)]
                else:
                    da_v, db_v = dbufs
                    d = da_v[pl.ds(g * 16, 16)] + db_v[pl.ds(g * 16, 16)]
                if softmax_eps:
                    inv16 = onev / (d + epsv)
                else:
                    inv16 = onev / jnp.maximum(d, onev)
                for l in range(LANES):
                    inv = inv16.at[iota * 0 + l].get(
                        mode="promise_in_bounds", unique_indices=False)
                    r = g * 16 + l
                    for j in range(DK):
                        sl = pl.ds(j * LANES, LANES)
                        a_v[r, sl] = (a_v[r, sl] + b_v[r, sl]) * inv
                return 0

            lax.fori_loop(0, nrows // 16, group, 0)
            pltpu.sync_copy(a_v.at[pl.ds(0, nrows)], o_hbm.at[pl.ds(base, nrows)])

        for t in range(ROUNDS):
            c = wid + NW * t
            if (t + 1) * NW <= FULLC:
                do(c * CH, CH)
            else:
                @pl.when(c < FULLC)
                def _():
                    do(c * CH, CH)
        if TREM:
            @pl.when(wid == TAILW)
            def _():
                do(FULLC * CH, TREM)

    return k


@functools.lru_cache(maxsize=None)
def _cell_gather_kernel(NCELL, N, D):
    CPW = (NCELL // NW) // 8 * 8
    REM = NCELL - CPW * NW
    CK = _chunk_1d(CPW)
    NCH = CPW // CK
    DK = D // LANES

    scratch = [
        pltpu.VMEM((CK, D), F32),
        pltpu.VMEM((CK, D), F32),
        pltpu.VMEM((CK, D), F32),
        pltpu.VMEM((CK, D), F32),
        pltpu.VMEM((CK,), jnp.int32),
        pltpu.VMEM((CK,), jnp.int32),
        pltpu.VMEM((CK,), jnp.int32),
        pltpu.SemaphoreType.DMA,
        pltpu.SemaphoreType.DMA,
        pltpu.SemaphoreType.DMA,
    ]
    if REM:
        scratch += [
            pltpu.VMEM((REM,), jnp.int32),
            pltpu.VMEM((REM,), jnp.int32),
            pltpu.VMEM((REM,), jnp.int32),
        ]

    @functools.partial(
        pl.kernel,
        out_type=jax.ShapeDtypeStruct((NCELL, D), F32),
        mesh=_mesh(),
        scratch_types=scratch,
    )
    def k(f0_hbm, f1_hbm, f2_hbm, nagg_hbm, o_hbm,
          g0_v, g1_v, g2_v, o_v, i0_v, i1_v, i2_v, s0, s1, s2, *tails):
        sid = lax.axis_index("s")
        cid = lax.axis_index("c")
        wid = sid * NC + cid
        third = jnp.full((LANES,), 1.0 / 3.0, F32)

        def do(b, n, i0, i1, i2):
            pltpu.sync_copy(f0_hbm.at[pl.ds(b, n)], i0)
            pltpu.sync_copy(f1_hbm.at[pl.ds(b, n)], i1)
            pltpu.sync_copy(f2_hbm.at[pl.ds(b, n)], i2)
            c0 = pltpu.async_copy(nagg_hbm.at[i0], g0_v.at[pl.ds(0, n)], s0)
            c1 = pltpu.async_copy(nagg_hbm.at[i1], g1_v.at[pl.ds(0, n)], s1)
            c2 = pltpu.async_copy(nagg_hbm.at[i2], g2_v.at[pl.ds(0, n)], s2)
            c0.wait()
            c1.wait()
            c2.wait()

            def cell(i, _):
                for j in range(DK):
                    sl = pl.ds(j * LANES, LANES)
                    o_v[i, sl] = (g0_v[i, sl] + g1_v[i, sl] + g2_v[i, sl]) * third
                return 0

            lax.fori_loop(0, n, cell, 0)
            pltpu.sync_copy(o_v.at[pl.ds(0, n)], o_hbm.at[pl.ds(b, n)])

        def chunk(c, _):
            do(wid * CPW + c * CK, CK, i0_v, i1_v, i2_v)
            return 0

        lax.fori_loop(0, NCH, chunk, 0)
        if REM:
            t0, t1, t2 = tails

            @pl.when(wid == NW - 1)
            def _():
                do(NW * CPW, REM, t0, t1, t2)

    return k


@functools.lru_cache(maxsize=None)
def _cell_scatter_kernel(NCELL, N, D):
    CPW = (NCELL // NW) // 8 * 8
    REM = NCELL - CPW * NW
    CK = _chunk_1d(CPW)
    NCH = CPW // CK
    DPT = (N // NS) // 8 * 8
    DREM = N - DPT * NS

    scratch = [
        pltpu.VMEM((CK, D), F32),
        pltpu.VMEM((CK,), jnp.int32),
        pltpu.VMEM((CK,), jnp.int32),
        pltpu.VMEM((CK,), jnp.int32),
        pltpu.VMEM((CK + 16,), jnp.int32),
        pltpu.VMEM((CK + 16,), jnp.int32),
        pltpu.VMEM((CK + 16,), jnp.int32),
        pltpu.VMEM((N + 16,), F32),
        pltpu.VMEM((DPT,), F32),
        pltpu.VMEM((DPT,), F32),
        pltpu.VMEM((16,), F32),
        pltpu.VMEM((16,), F32),
        pltpu.VMEM_SHARED((N, D), F32),
        pltpu.VMEM_SHARED((NS * N,), F32),
    ]
    if REM:
        scratch += [
            pltpu.VMEM((REM,), jnp.int32),
            pltpu.VMEM((REM,), jnp.int32),
            pltpu.VMEM((REM,), jnp.int32),
            pltpu.VMEM((REM + 16,), jnp.int32),
            pltpu.VMEM((REM + 16,), jnp.int32),
            pltpu.VMEM((REM + 16,), jnp.int32),
        ]

    @functools.partial(
        pl.kernel,
        out_type=(
            jax.ShapeDtypeStruct((NC * N, D), F32),
            jax.ShapeDtypeStruct((NC * N,), F32),
        ),
        mesh=_mesh(),
        scratch_types=scratch,
    )
    def k(cell_hbm, f0_hbm, f1_hbm, f2_hbm, zn_hbm, s_out, c_out,
          buf_v, i0_v, i1_v, i2_v, i0p_v, i1p_v, i2p_v, cnt_v,
          acc_v, tmp_v, ta_v, tt_v, s_sh, cnt_st, *tails):
        sid = lax.axis_index("s")
        cid = lax.axis_index("c")
        wid = sid * NC + cid
        iota = lax.iota(jnp.int32, LANES)
        zeros16 = jnp.zeros((LANES,), F32)
        onev = jnp.full((LANES,), 1.0, F32)
        eye0 = jnp.where(iota == 0, onev, zeros16)

        def zc(i, _):
            cnt_v[pl.ds(i * 16, 16)] = zeros16
            return 0

        lax.fori_loop(0, (N + 16) // 16, zc, 0)
        zoff = sid * DPT
        pltpu.sync_copy(zn_hbm.at[pl.ds(zoff, DPT)], s_sh.at[pl.ds(zoff, DPT)])
        if DREM:
            @pl.when(sid == NS - 1)
            def _():
                pltpu.sync_copy(zn_hbm.at[pl.ds(NS * DPT, DREM)],
                                s_sh.at[pl.ds(NS * DPT, DREM)])
        plsc.subcore_barrier()

        def do(b, n, i0, i1, i2, i0p, i1p, i2p):
            pltpu.sync_copy(cell_hbm.at[pl.ds(b, n)], buf_v.at[pl.ds(0, n)])
            pltpu.sync_copy(f0_hbm.at[pl.ds(b, n)], i0)
            pltpu.sync_copy(f1_hbm.at[pl.ds(b, n)], i1)
            pltpu.sync_copy(f2_hbm.at[pl.ds(b, n)], i2)
            pltpu.sync_copy(f0_hbm.at[pl.ds(b, n)], i0p.at[pl.ds(0, n)])
            pltpu.sync_copy(f1_hbm.at[pl.ds(b, n)], i1p.at[pl.ds(0, n)])
            pltpu.sync_copy(f2_hbm.at[pl.ds(b, n)], i2p.at[pl.ds(0, n)])
            pltpu.sync_copy(buf_v.at[pl.ds(0, n)], s_sh.at[i0], add=True)
            pltpu.sync_copy(buf_v.at[pl.ds(0, n)], s_sh.at[i1], add=True)
            pltpu.sync_copy(buf_v.at[pl.ds(0, n)], s_sh.at[i2], add=True)

            def cellcnt(i, _):
                for ipref in (i0p, i1p, i2p):
                    nn = ipref[pl.ds(i, 16)][0]
                    cnt_v[pl.ds(nn, 16)] = cnt_v[pl.ds(nn, 16)] + eye0
                return 0

            lax.fori_loop(0, n, cellcnt, 0)

        def chunk(c, _):
            do(wid * CPW + c * CK, CK, i0_v, i1_v, i2_v, i0p_v, i1p_v, i2p_v)
            return 0

        lax.fori_loop(0, NCH, chunk, 0)
        if REM:
            t0, t1, t2, t0p, t1p, t2p = tails

            @pl.when(wid == NW - 1)
            def _():
                do(NW * CPW, REM, t0, t1, t2, t0p, t1p, t2p)

        pltpu.sync_copy(cnt_v.at[pl.ds(0, N)], cnt_st.at[pl.ds(sid * N, N)])
        plsc.subcore_barrier()
        pltpu.sync_copy(s_sh.at[pl.ds(zoff, DPT)],
                        s_out.at[pl.ds(cid * N + zoff, DPT)])
        if DREM:
            @pl.when(sid == NS - 1)
            def _():
                pltpu.sync_copy(s_sh.at[pl.ds(NS * DPT, DREM)],
                                s_out.at[pl.ds(cid * N + NS * DPT, DREM)])
        _tile_reduce_dump(cnt_st, acc_v, tmp_v, ta_v, tt_v, c_out,
                          N, DPT, DREM, sid, cid)

    return k


def _matmul(cell_attr, cell_agg, w1, w2, b8):
    M, D = cell_attr.shape
    BM = 2000
    assert M % BM == 0

    def body(a_ref, g_ref, w1_ref, w2_ref, b_ref, o_ref):
        o_ref[...] = (
            jnp.dot(a_ref[...], w1_ref[...], preferred_element_type=F32)
            + jnp.dot(g_ref[...], w2_ref[...], preferred_element_type=F32)
            + b_ref[0:1, :]
        )

    return pl.pallas_call(
        body,
        grid=(M // BM,),
        in_specs=[
            pl.BlockSpec((BM, D), lambda i: (i, 0)),
            pl.BlockSpec((BM, D), lambda i: (i, 0)),
            pl.BlockSpec((D, D), lambda i: (0, 0)),
            pl.BlockSpec((D, D), lambda i: (0, 0)),
            pl.BlockSpec((8, D), lambda i: (0, 0)),
        ],
        out_specs=pl.BlockSpec((BM, D), lambda i: (i, 0)),
        out_shape=jax.ShapeDtypeStruct((M, D), F32),
    )(cell_attr, cell_agg, w1, w2, b8)


def kernel(cell_attr, edge_attr, node_embedding, edge_index, face, W, b):
    E, D = edge_attr.shape
    N = node_embedding.shape[0]
    NCELL = cell_attr.shape[0]

    senders = edge_index[0]
    receivers = edge_index[1]
    f0, f1, f2 = face[0], face[1], face[2]

    zn = jnp.zeros((N, D), F32)
    u_parts, den_parts = _edge_kernel(E, N, D)(
        edge_attr, senders, receivers, node_embedding, zn)
    npad = ((N + 127) // 128) * 128
    node_agg = _norm_kernel(N, D, True, True)(
        u_parts, den_parts.reshape(NW, npad))
    cell_agg = _cell_gather_kernel(NCELL, N, D)(f0, f1, f2, node_agg)
    b8 = jnp.broadcast_to(b.reshape(1, D), (8, D))
    cell_new = _matmul(cell_attr, cell_agg, W[:D], W[D:], b8)
    s_parts, c_parts = _cell_scatter_kernel(NCELL, N, D)(
        cell_new, f0, f1, f2, zn)
    node_attr = _norm_kernel(N, D, False, False)(s_parts, c_parts)
    return cell_new, node_attr


# speedup vs baseline: 18.5915x; 1.0671x over previous
import functools

import jax
import jax.numpy as jnp
from jax import lax
from jax.experimental import pallas as pl
from jax.experimental.pallas import tpu as pltpu
from jax.experimental.pallas import tpu_sc as plsc

NC = 2
NS = 16
NW = NC * NS
LANES = 16

F32 = jnp.float32

def _mesh():
    return plsc.VectorSubcoreMesh(
        core_axis_name="c", subcore_axis_name="s", num_cores=NC, num_subcores=NS
    )


def _chunk_1d(n, cap=128):
    for k in range(cap, 0, -8):
        if n % k == 0:
            return k
    raise ValueError(f"no 8-aligned chunk for {n}")


def _hsum(x):
    iota = lax.iota(jnp.int32, LANES)
    for m in (1, 2, 4, 8):
        x = x + x.at[iota ^ m].get(mode="promise_in_bounds", unique_indices=True)
    return x


def _windows(n):
    wins = [(s, 0) for s in range(0, n - 15, 16)]
    cov = 16 * len(wins)
    if cov < n:
        wins.append((n - 16, 16 - (n - cov)))
    return wins


def _tile_reduce_dump(src_sh, acc_v, tmp_v, ta_v, tt_v, out_hbm, N, DPT, DREM,
                      sid, cid):
    zoff = sid * DPT
    pltpu.sync_copy(src_sh.at[pl.ds(zoff, DPT)], acc_v)
    for w in range(1, NS):
        pltpu.sync_copy(src_sh.at[pl.ds(w * N + zoff, DPT)], tmp_v)

        def addw(i, _):
            acc_v[pl.ds(i * 16, 16)] = (acc_v[pl.ds(i * 16, 16)]
                                        + tmp_v[pl.ds(i * 16, 16)])
            return 0

        lax.fori_loop(0, DPT // 16, addw, 0)
    pltpu.sync_copy(acc_v, out_hbm.at[pl.ds(cid * N + zoff, DPT)])
    if DREM:
        @pl.when(sid == NS - 1)
        def _():
            pltpu.sync_copy(src_sh.at[pl.ds(NS * DPT, DREM)], ta_v)
            for w in range(1, NS):
                pltpu.sync_copy(src_sh.at[pl.ds(w * N + NS * DPT, DREM)], tt_v)
                ta_v[...] = ta_v[...] + tt_v[...]
            pltpu.sync_copy(ta_v, out_hbm.at[pl.ds(cid * N + NS * DPT, DREM)])


@functools.lru_cache(maxsize=None)
def _edge_kernel(E, N, D):
    EPW = E // NW
    EK = _chunk_1d(EPW, cap=40)
    NCH = EPW // EK
    DPT = (N // NS) // 8 * 8
    DREM = N - DPT * NS
    DK = D // LANES
    inv_scale = 1.0 / float(D) ** 0.5
    NPAD = ((N + 127) // 128) * 128
    assert N % 16 == 0 and NCH % 2 == 0 and EK % 4 == 0

    @functools.partial(
        pl.kernel,
        out_type=(
            jax.ShapeDtypeStruct((NC * N, D), F32),
            jax.ShapeDtypeStruct((NW * NPAD,), F32),
        ),
        mesh=_mesh(),
        scratch_types=[
            pltpu.VMEM((EK, D), F32), pltpu.VMEM((EK, D), F32),
            pltpu.VMEM((EK, D), F32), pltpu.VMEM((EK, D), F32),
            pltpu.VMEM((EK, D), F32), pltpu.VMEM((EK, D), F32),
            pltpu.VMEM((EK,), jnp.int32), pltpu.VMEM((EK,), jnp.int32),
            pltpu.VMEM((EK,), jnp.int32), pltpu.VMEM((EK,), jnp.int32),
            pltpu.VMEM((EK + 16,), jnp.int32), pltpu.VMEM((EK + 16,), jnp.int32),
            pltpu.VMEM((EK + 16,), jnp.int32), pltpu.VMEM((EK + 16,), jnp.int32),
            pltpu.VMEM((N + 16,), F32),
            pltpu.VMEM_SHARED((N, D), F32),
            pltpu.SemaphoreType.DMA, pltpu.SemaphoreType.DMA,
            pltpu.SemaphoreType.DMA, pltpu.SemaphoreType.DMA,
            pltpu.SemaphoreType.DMA, pltpu.SemaphoreType.DMA,
        ],
    )
    def k1(edge_hbm, snd_hbm, rcv_hbm, nemb_hbm, zn_hbm, u_out, den_out,
           e_v0, e_v1, nr_v0, nr_v1, ns_v0, ns_v1, ir_v0, ir_v1,
           is_v0, is_v1, irp_v0, irp_v1, isp_v0, isp_v1, den_v, u_sh,
           sl0, sl1, sgr0, sgr1, sgs0, sgs1):
        sid = lax.axis_index("s")
        cid = lax.axis_index("c")
        wid = sid * NC + cid
        iota = lax.iota(jnp.int32, LANES)
        zeros16 = jnp.zeros((LANES,), F32)
        onev = jnp.full((LANES,), 1.0, F32)
        eye0 = jnp.where(iota == 0, onev, zeros16)
        inv_scale_v = jnp.full((LANES,), inv_scale, F32)
        ebase = wid * EPW

        bufs = (
            (e_v0, nr_v0, ns_v0, ir_v0, is_v0, irp_v0, isp_v0, sl0, sgr0, sgs0),
            (e_v1, nr_v1, ns_v1, ir_v1, is_v1, irp_v1, isp_v1, sl1, sgr1, sgs1),
        )

        def zden(i, _):
            den_v[pl.ds(i * 16, 16)] = zeros16
            return 0

        lax.fori_loop(0, (N + 16) // 16, zden, 0)
        zoff = sid * DPT
        pltpu.sync_copy(zn_hbm.at[pl.ds(zoff, DPT)], u_sh.at[pl.ds(zoff, DPT)])
        if DREM:
            @pl.when(sid == NS - 1)
            def _():
                pltpu.sync_copy(zn_hbm.at[pl.ds(NS * DPT, DREM)],
                                u_sh.at[pl.ds(NS * DPT, DREM)])
        plsc.subcore_barrier()

        def load_list(c, B):
            e_v, nr_v, ns_v, ir_v, is_v, irp_v, isp_v = B[:7]
            b = ebase + c * EK
            return [
                (snd_hbm.at[pl.ds(b, EK)], is_v),
                (rcv_hbm.at[pl.ds(b, EK)], ir_v),
                (snd_hbm.at[pl.ds(b, EK)], isp_v.at[pl.ds(0, EK)]),
                (rcv_hbm.at[pl.ds(b, EK)], irp_v.at[pl.ds(0, EK)]),
                (edge_hbm.at[pl.ds(b, EK)], e_v),
            ]

        def issue_loads(c, B):
            for s, d in load_list(c, B):
                pltpu.async_copy(s, d, B[7])

        def wait_loads(c, B):
            for s, d in load_list(c, B):
                pltpu.make_async_copy(s, d, B[7]).wait()

        def issue_gathers(B):
            pltpu.async_copy(nemb_hbm.at[B[3]], B[1], B[8])
            pltpu.async_copy(nemb_hbm.at[B[4]], B[2], B[9])

        def wait_gathers(B):
            pltpu.make_async_copy(nemb_hbm.at[B[3]], B[1], B[8]).wait()
            pltpu.make_async_copy(nemb_hbm.at[B[4]], B[2], B[9]).wait()

        def compute_scatter(B):
            e_v, nr_v, ns_v, ir_v, is_v, irp_v, isp_v = B[:7]

            def edge(i2, _):
                exs_l = []
                for u in range(4):
                    i = i2 * 4 + u
                    es = [e_v[i, pl.ds(j * LANES, LANES)] for j in range(DK)]
                    accr = es[0] * nr_v[i, pl.ds(0, LANES)]
                    accs = es[0] * ns_v[i, pl.ds(0, LANES)]
                    for j in range(1, DK):
                        accr = accr + es[j] * nr_v[i, pl.ds(j * LANES, LANES)]
                        accs = accs + es[j] * ns_v[i, pl.ds(j * LANES, LANES)]
                    exr = jnp.exp(_hsum(accr) * inv_scale_v)
                    exs = jnp.exp(_hsum(accs) * inv_scale_v)
                    for j in range(DK):
                        nr_v[i, pl.ds(j * LANES, LANES)] = es[j] * exr
                        ns_v[i, pl.ds(j * LANES, LANES)] = es[j] * exs
                    exs_l.append((i, exr, exs))
                for (i, exr, exs) in exs_l:
                    nri = irp_v[pl.ds(i, 16)][0]
                    den_v[pl.ds(nri, 16)] = den_v[pl.ds(nri, 16)] + exr * eye0
                    nsi = isp_v[pl.ds(i, 16)][0]
                    den_v[pl.ds(nsi, 16)] = den_v[pl.ds(nsi, 16)] + exs * eye0
                return 0

            lax.fori_loop(0, EK // 4, edge, 0)
            pltpu.sync_copy(nr_v, u_sh.at[ir_v], add=True)
            pltpu.sync_copy(ns_v, u_sh.at[is_v], add=True)

        issue_loads(0, bufs[0])
        wait_loads(0, bufs[0])
        issue_gathers(bufs[0])
        if NCH > 1:
            issue_loads(1, bufs[1])

        def pair(p, _):
            for bpar in (0, 1):
                c = 2 * p + bpar
                B = bufs[bpar]
                Bn = bufs[1 - bpar]

                @pl.when(c + 1 < NCH)
                def _():
                    wait_loads(c + 1, Bn)
                    issue_gathers(Bn)

                wait_gathers(B)
                compute_scatter(B)

                @pl.when(c + 2 < NCH)
                def _():
                    issue_loads(c + 2, B)
            return 0

        lax.fori_loop(0, NCH // 2, pair, 0)
        plsc.subcore_barrier()

        pltpu.sync_copy(u_sh.at[pl.ds(zoff, DPT)],
                        u_out.at[pl.ds(cid * N + zoff, DPT)])
        if DREM:
            @pl.when(sid == NS - 1)
            def _():
                pltpu.sync_copy(u_sh.at[pl.ds(NS * DPT, DREM)],
                                u_out.at[pl.ds(cid * N + NS * DPT, DREM)])
        pltpu.sync_copy(den_v.at[pl.ds(0, N)],
                        den_out.at[pl.ds(wid * NPAD, N)])

    return k1


@functools.lru_cache(maxsize=None)
def _norm_kernel(N, D, softmax_eps, many_parts):
    CH = 128
    FULLC = N // CH
    TREM = N - FULLC * CH
    TAILW = FULLC % NW
    ROUNDS = (FULLC + NW - 1) // NW
    DK = D // LANES
    NPAD = ((N + 127) // 128) * 128
    assert TREM % 16 == 0

    scratch = [
        pltpu.VMEM((CH, D), F32),
        pltpu.VMEM((CH, D), F32),
    ]
    if many_parts:
        scratch.append(pltpu.VMEM((NW, CH), F32))
    else:
        scratch += [pltpu.VMEM((CH,), F32), pltpu.VMEM((CH,), F32)]

    @functools.partial(
        pl.kernel,
        out_type=jax.ShapeDtypeStruct((N, D), F32),
        mesh=_mesh(),
        scratch_types=scratch,
    )
    def k(x_hbm, d_hbm, o_hbm, a_v, b_v, *dbufs):
        sid = lax.axis_index("s")
        cid = lax.axis_index("c")
        wid = sid * NC + cid
        iota = lax.iota(jnp.int32, LANES)
        onev = jnp.full((LANES,), 1.0, F32)
        epsv = jnp.full((LANES,), 1e-16, F32)

        def do(base, nrows):
            pltpu.sync_copy(x_hbm.at[pl.ds(base, nrows)], a_v.at[pl.ds(0, nrows)])
            pltpu.sync_copy(x_hbm.at[pl.ds(N + base, nrows)], b_v.at[pl.ds(0, nrows)])
            if many_parts:
                dbuf = dbufs[0]
                pltpu.sync_copy(d_hbm.at[:, pl.ds(base, CH)], dbuf)
            else:
                da_v, db_v = dbufs
                pltpu.sync_copy(d_hbm.at[pl.ds(base, nrows)], da_v.at[pl.ds(0, nrows)])
                pltpu.sync_copy(d_hbm.at[pl.ds(N + base, nrows)], db_v.at[pl.ds(0, nrows)])

            def group(g, _):
                if many_parts:
                    dbuf = dbufs[0]
                    d = dbuf[0, pl.ds(g * 16, 16)]
                    for w in range(1, NW):
                        d = d + dbuf[w, pl.ds(g * 16, 16)]
                else:
                    da_v, db_v = dbufs
                    d = da_v[pl.ds(g * 16, 16)] + db_v[pl.ds(g * 16, 16)]
                if softmax_eps:
                    inv16 = onev / (d + epsv)
                else:
                    inv16 = onev / jnp.maximum(d, onev)
                for l in range(LANES):
                    inv = inv16.at[iota * 0 + l].get(
                        mode="promise_in_bounds", unique_indices=False)
                    r = g * 16 + l
                    for j in range(DK):
                        sl = pl.ds(j * LANES, LANES)
                        a_v[r, sl] = (a_v[r, sl] + b_v[r, sl]) * inv
                return 0

            lax.fori_loop(0, nrows // 16, group, 0)
            pltpu.sync_copy(a_v.at[pl.ds(0, nrows)], o_hbm.at[pl.ds(base, nrows)])

        for t in range(ROUNDS):
            c = wid + NW * t
            if (t + 1) * NW <= FULLC:
                do(c * CH, CH)
            else:
                @pl.when(c < FULLC)
                def _():
                    do(c * CH, CH)
        if TREM:
            @pl.when(wid == TAILW)
            def _():
                do(FULLC * CH, TREM)

    return k


@functools.lru_cache(maxsize=None)
def _cell_gather_kernel(NCELL, N, D):
    CPW = (NCELL // NW) // 8 * 8
    REM = NCELL - CPW * NW
    CK = _chunk_1d(CPW)
    NCH = CPW // CK
    DK = D // LANES

    scratch = [
        pltpu.VMEM((CK, D), F32),
        pltpu.VMEM((CK, D), F32),
        pltpu.VMEM((CK, D), F32),
        pltpu.VMEM((CK, D), F32),
        pltpu.VMEM((CK,), jnp.int32),
        pltpu.VMEM((CK,), jnp.int32),
        pltpu.VMEM((CK,), jnp.int32),
        pltpu.SemaphoreType.DMA,
        pltpu.SemaphoreType.DMA,
        pltpu.SemaphoreType.DMA,
    ]
    if REM:
        scratch += [
            pltpu.VMEM((REM,), jnp.int32),
            pltpu.VMEM((REM,), jnp.int32),
            pltpu.VMEM((REM,), jnp.int32),
        ]

    @functools.partial(
        pl.kernel,
        out_type=jax.ShapeDtypeStruct((NCELL, D), F32),
        mesh=_mesh(),
        scratch_types=scratch,
    )
    def k(f0_hbm, f1_hbm, f2_hbm, nagg_hbm, o_hbm,
          g0_v, g1_v, g2_v, o_v, i0_v, i1_v, i2_v, s0, s1, s2, *tails):
        sid = lax.axis_index("s")
        cid = lax.axis_index("c")
        wid = sid * NC + cid
        third = jnp.full((LANES,), 1.0 / 3.0, F32)

        def do(b, n, i0, i1, i2):
            pltpu.sync_copy(f0_hbm.at[pl.ds(b, n)], i0)
            pltpu.sync_copy(f1_hbm.at[pl.ds(b, n)], i1)
            pltpu.sync_copy(f2_hbm.at[pl.ds(b, n)], i2)
            c0 = pltpu.async_copy(nagg_hbm.at[i0], g0_v.at[pl.ds(0, n)], s0)
            c1 = pltpu.async_copy(nagg_hbm.at[i1], g1_v.at[pl.ds(0, n)], s1)
            c2 = pltpu.async_copy(nagg_hbm.at[i2], g2_v.at[pl.ds(0, n)], s2)
            c0.wait()
            c1.wait()
            c2.wait()

            def cell(i, _):
                for j in range(DK):
                    sl = pl.ds(j * LANES, LANES)
                    o_v[i, sl] = (g0_v[i, sl] + g1_v[i, sl] + g2_v[i, sl]) * third
                return 0

            lax.fori_loop(0, n, cell, 0)
            pltpu.sync_copy(o_v.at[pl.ds(0, n)], o_hbm.at[pl.ds(b, n)])

        def chunk(c, _):
            do(wid * CPW + c * CK, CK, i0_v, i1_v, i2_v)
            return 0

        lax.fori_loop(0, NCH, chunk, 0)
        if REM:
            t0, t1, t2 = tails

            @pl.when(wid == NW - 1)
            def _():
                do(NW * CPW, REM, t0, t1, t2)

    return k


@functools.lru_cache(maxsize=None)
def _cell_scatter_kernel(NCELL, N, D):
    CPW = (NCELL // NW) // 8 * 8
    REM = NCELL - CPW * NW
    CK = _chunk_1d(CPW)
    NCH = CPW // CK
    DPT = (N // NS) // 8 * 8
    DREM = N - DPT * NS

    scratch = [
        pltpu.VMEM((CK, D), F32),
        pltpu.VMEM((CK,), jnp.int32),
        pltpu.VMEM((CK,), jnp.int32),
        pltpu.VMEM((CK,), jnp.int32),
        pltpu.VMEM((CK + 16,), jnp.int32),
        pltpu.VMEM((CK + 16,), jnp.int32),
        pltpu.VMEM((CK + 16,), jnp.int32),
        pltpu.VMEM((N + 16,), F32),
        pltpu.VMEM((DPT,), F32),
        pltpu.VMEM((DPT,), F32),
        pltpu.VMEM((16,), F32),
        pltpu.VMEM((16,), F32),
        pltpu.VMEM_SHARED((N, D), F32),
        pltpu.VMEM_SHARED((NS * N,), F32),
    ]
    if REM:
        scratch += [
            pltpu.VMEM((REM,), jnp.int32),
            pltpu.VMEM((REM,), jnp.int32),
            pltpu.VMEM((REM,), jnp.int32),
            pltpu.VMEM((REM + 16,), jnp.int32),
            pltpu.VMEM((REM + 16,), jnp.int32),
            pltpu.VMEM((REM + 16,), jnp.int32),
        ]

    @functools.partial(
        pl.kernel,
        out_type=(
            jax.ShapeDtypeStruct((NC * N, D), F32),
            jax.ShapeDtypeStruct((NC * N,), F32),
        ),
        mesh=_mesh(),
        scratch_types=scratch,
    )
    def k(cell_hbm, f0_hbm, f1_hbm, f2_hbm, zn_hbm, s_out, c_out,
          buf_v, i0_v, i1_v, i2_v, i0p_v, i1p_v, i2p_v, cnt_v,
          acc_v, tmp_v, ta_v, tt_v, s_sh, cnt_st, *tails):
        sid = lax.axis_index("s")
        cid = lax.axis_index("c")
        wid = sid * NC + cid
        iota = lax.iota(jnp.int32, LANES)
        zeros16 = jnp.zeros((LANES,), F32)
        onev = jnp.full((LANES,), 1.0, F32)
        eye0 = jnp.where(iota == 0, onev, zeros16)

        def zc(i, _):
            cnt_v[pl.ds(i * 16, 16)] = zeros16
            return 0

        lax.fori_loop(0, (N + 16) // 16, zc, 0)
        zoff = sid * DPT
        pltpu.sync_copy(zn_hbm.at[pl.ds(zoff, DPT)], s_sh.at[pl.ds(zoff, DPT)])
        if DREM:
            @pl.when(sid == NS - 1)
            def _():
                pltpu.sync_copy(zn_hbm.at[pl.ds(NS * DPT, DREM)],
                                s_sh.at[pl.ds(NS * DPT, DREM)])
        plsc.subcore_barrier()

        def do(b, n, i0, i1, i2, i0p, i1p, i2p):
            pltpu.sync_copy(cell_hbm.at[pl.ds(b, n)], buf_v.at[pl.ds(0, n)])
            pltpu.sync_copy(f0_hbm.at[pl.ds(b, n)], i0)
            pltpu.sync_copy(f1_hbm.at[pl.ds(b, n)], i1)
            pltpu.sync_copy(f2_hbm.at[pl.ds(b, n)], i2)
            pltpu.sync_copy(f0_hbm.at[pl.ds(b, n)], i0p.at[pl.ds(0, n)])
            pltpu.sync_copy(f1_hbm.at[pl.ds(b, n)], i1p.at[pl.ds(0, n)])
            pltpu.sync_copy(f2_hbm.at[pl.ds(b, n)], i2p.at[pl.ds(0, n)])
            pltpu.sync_copy(buf_v.at[pl.ds(0, n)], s_sh.at[i0], add=True)
            pltpu.sync_copy(buf_v.at[pl.ds(0, n)], s_sh.at[i1], add=True)
            pltpu.sync_copy(buf_v.at[pl.ds(0, n)], s_sh.at[i2], add=True)

            def cellcnt(i, _):
                for ipref in (i0p, i1p, i2p):
                    nn = ipref[pl.ds(i, 16)][0]
                    cnt_v[pl.ds(nn, 16)] = cnt_v[pl.ds(nn, 16)] + eye0
                return 0

            lax.fori_loop(0, n, cellcnt, 0)

        def chunk(c, _):
            do(wid * CPW + c * CK, CK, i0_v, i1_v, i2_v, i0p_v, i1p_v, i2p_v)
            return 0

        lax.fori_loop(0, NCH, chunk, 0)
        if REM:
            t0, t1, t2, t0p, t1p, t2p = tails

            @pl.when(wid == NW - 1)
            def _():
                do(NW * CPW, REM, t0, t1, t2, t0p, t1p, t2p)

        pltpu.sync_copy(cnt_v.at[pl.ds(0, N)], cnt_st.at[pl.ds(sid * N, N)])
        plsc.subcore_barrier()
        pltpu.sync_copy(s_sh.at[pl.ds(zoff, DPT)],
                        s_out.at[pl.ds(cid * N + zoff, DPT)])
        if DREM:
            @pl.when(sid == NS - 1)
            def _():
                pltpu.sync_copy(s_sh.at[pl.ds(NS * DPT, DREM)],
                                s_out.at[pl.ds(cid * N + NS * DPT, DREM)])
        _tile_reduce_dump(cnt_st, acc_v, tmp_v, ta_v, tt_v, c_out,
                          N, DPT, DREM, sid, cid)

    return k


def _matmul(cell_attr, cell_agg, w1, w2, b8):
    M, D = cell_attr.shape
    BM = 2000
    assert M % BM == 0

    def body(a_ref, g_ref, w1_ref, w2_ref, b_ref, o_ref):
        o_ref[...] = (
            jnp.dot(a_ref[...], w1_ref[...], preferred_element_type=F32)
            + jnp.dot(g_ref[...], w2_ref[...], preferred_element_type=F32)
            + b_ref[0:1, :]
        )

    return pl.pallas_call(
        body,
        grid=(M // BM,),
        in_specs=[
            pl.BlockSpec((BM, D), lambda i: (i, 0)),
            pl.BlockSpec((BM, D), lambda i: (i, 0)),
            pl.BlockSpec((D, D), lambda i: (0, 0)),
            pl.BlockSpec((D, D), lambda i: (0, 0)),
            pl.BlockSpec((8, D), lambda i: (0, 0)),
        ],
        out_specs=pl.BlockSpec((BM, D), lambda i: (i, 0)),
        out_shape=jax.ShapeDtypeStruct((M, D), F32),
    )(cell_attr, cell_agg, w1, w2, b8)


def kernel(cell_attr, edge_attr, node_embedding, edge_index, face, W, b):
    E, D = edge_attr.shape
    N = node_embedding.shape[0]
    NCELL = cell_attr.shape[0]

    senders = edge_index[0]
    receivers = edge_index[1]
    f0, f1, f2 = face[0], face[1], face[2]

    zn = jnp.zeros((N, D), F32)
    u_parts, den_parts = _edge_kernel(E, N, D)(
        edge_attr, senders, receivers, node_embedding, zn)
    npad = ((N + 127) // 128) * 128
    node_agg = _norm_kernel(N, D, True, True)(
        u_parts, den_parts.reshape(NW, npad))
    cell_agg = _cell_gather_kernel(NCELL, N, D)(f0, f1, f2, node_agg)
    b8 = jnp.broadcast_to(b.reshape(1, D), (8, D))
    cell_new = _matmul(cell_attr, cell_agg, W[:D], W[D:], b8)
    s_parts, c_parts = _cell_scatter_kernel(NCELL, N, D)(
        cell_new, f0, f1, f2, zn)
    node_attr = _norm_kernel(N, D, False, False)(s_parts, c_parts)
    return cell_new, node_attr


# speedup vs baseline: 18.7428x; 1.0081x over previous
import functools

import jax
import jax.numpy as jnp
from jax import lax
from jax.experimental import pallas as pl
from jax.experimental.pallas import tpu as pltpu
from jax.experimental.pallas import tpu_sc as plsc

NC = 2
NS = 16
NW = NC * NS
LANES = 16

F32 = jnp.float32

def _mesh():
    return plsc.VectorSubcoreMesh(
        core_axis_name="c", subcore_axis_name="s", num_cores=NC, num_subcores=NS
    )


def _chunk_1d(n, cap=128):
    for k in range(cap, 0, -8):
        if n % k == 0:
            return k
    raise ValueError(f"no 8-aligned chunk for {n}")


def _hsum(x):
    iota = lax.iota(jnp.int32, LANES)
    for m in (1, 2, 4, 8):
        x = x + x.at[iota ^ m].get(mode="promise_in_bounds", unique_indices=True)
    return x


def _windows(n):
    wins = [(s, 0) for s in range(0, n - 15, 16)]
    cov = 16 * len(wins)
    if cov < n:
        wins.append((n - 16, 16 - (n - cov)))
    return wins


def _tile_reduce_dump(src_sh, acc_v, tmp_v, ta_v, tt_v, out_hbm, N, DPT, DREM,
                      sid, cid):
    zoff = sid * DPT
    pltpu.sync_copy(src_sh.at[pl.ds(zoff, DPT)], acc_v)
    for w in range(1, NS):
        pltpu.sync_copy(src_sh.at[pl.ds(w * N + zoff, DPT)], tmp_v)

        def addw(i, _):
            acc_v[pl.ds(i * 16, 16)] = (acc_v[pl.ds(i * 16, 16)]
                                        + tmp_v[pl.ds(i * 16, 16)])
            return 0

        lax.fori_loop(0, DPT // 16, addw, 0)
    pltpu.sync_copy(acc_v, out_hbm.at[pl.ds(cid * N + zoff, DPT)])
    if DREM:
        @pl.when(sid == NS - 1)
        def _():
            pltpu.sync_copy(src_sh.at[pl.ds(NS * DPT, DREM)], ta_v)
            for w in range(1, NS):
                pltpu.sync_copy(src_sh.at[pl.ds(w * N + NS * DPT, DREM)], tt_v)
                ta_v[...] = ta_v[...] + tt_v[...]
            pltpu.sync_copy(ta_v, out_hbm.at[pl.ds(cid * N + NS * DPT, DREM)])


@functools.lru_cache(maxsize=None)
def _edge_kernel(E, N, D):
    EPW = E // NW
    EK = _chunk_1d(EPW, cap=40)
    NCH = EPW // EK
    DPT = (N // NS) // 8 * 8
    DREM = N - DPT * NS
    DK = D // LANES
    inv_scale = 1.0 / float(D) ** 0.5
    NPAD = ((N + 127) // 128) * 128
    assert N % 16 == 0 and NCH % 2 == 0 and EK % 8 == 0

    @functools.partial(
        pl.kernel,
        out_type=(
            jax.ShapeDtypeStruct((NC * N, D), F32),
            jax.ShapeDtypeStruct((NW * NPAD,), F32),
        ),
        mesh=_mesh(),
        scratch_types=[
            pltpu.VMEM((EK, D), F32), pltpu.VMEM((EK, D), F32),
            pltpu.VMEM((EK, D), F32), pltpu.VMEM((EK, D), F32),
            pltpu.VMEM((EK, D), F32), pltpu.VMEM((EK, D), F32),
            pltpu.VMEM((EK,), jnp.int32), pltpu.VMEM((EK,), jnp.int32),
            pltpu.VMEM((EK,), jnp.int32), pltpu.VMEM((EK,), jnp.int32),
            pltpu.VMEM((EK + 16,), jnp.int32), pltpu.VMEM((EK + 16,), jnp.int32),
            pltpu.VMEM((EK + 16,), jnp.int32), pltpu.VMEM((EK + 16,), jnp.int32),
            pltpu.VMEM((N + 16,), F32),
            pltpu.VMEM_SHARED((N, D), F32),
            pltpu.SemaphoreType.DMA, pltpu.SemaphoreType.DMA,
            pltpu.SemaphoreType.DMA, pltpu.SemaphoreType.DMA,
            pltpu.SemaphoreType.DMA, pltpu.SemaphoreType.DMA,
        ],
    )
    def k1(edge_hbm, snd_hbm, rcv_hbm, nemb_hbm, zn_hbm, u_out, den_out,
           e_v0, e_v1, nr_v0, nr_v1, ns_v0, ns_v1, ir_v0, ir_v1,
           is_v0, is_v1, irp_v0, irp_v1, isp_v0, isp_v1, den_v, u_sh,
           sl0, sl1, sgr0, sgr1, sgs0, sgs1):
        sid = lax.axis_index("s")
        cid = lax.axis_index("c")
        wid = sid * NC + cid
        iota = lax.iota(jnp.int32, LANES)
        zeros16 = jnp.zeros((LANES,), F32)
        onev = jnp.full((LANES,), 1.0, F32)
        eye0 = jnp.where(iota == 0, onev, zeros16)
        inv_scale_v = jnp.full((LANES,), inv_scale, F32)
        ebase = wid * EPW

        bufs = (
            (e_v0, nr_v0, ns_v0, ir_v0, is_v0, irp_v0, isp_v0, sl0, sgr0, sgs0),
            (e_v1, nr_v1, ns_v1, ir_v1, is_v1, irp_v1, isp_v1, sl1, sgr1, sgs1),
        )

        def zden(i, _):
            den_v[pl.ds(i * 16, 16)] = zeros16
            return 0

        lax.fori_loop(0, (N + 16) // 16, zden, 0)
        zoff = sid * DPT
        pltpu.sync_copy(zn_hbm.at[pl.ds(zoff, DPT)], u_sh.at[pl.ds(zoff, DPT)])
        if DREM:
            @pl.when(sid == NS - 1)
            def _():
                pltpu.sync_copy(zn_hbm.at[pl.ds(NS * DPT, DREM)],
                                u_sh.at[pl.ds(NS * DPT, DREM)])
        plsc.subcore_barrier()

        def load_list(c, B):
            e_v, nr_v, ns_v, ir_v, is_v, irp_v, isp_v = B[:7]
            b = ebase + c * EK
            return [
                (snd_hbm.at[pl.ds(b, EK)], is_v),
                (rcv_hbm.at[pl.ds(b, EK)], ir_v),
                (snd_hbm.at[pl.ds(b, EK)], isp_v.at[pl.ds(0, EK)]),
                (rcv_hbm.at[pl.ds(b, EK)], irp_v.at[pl.ds(0, EK)]),
                (edge_hbm.at[pl.ds(b, EK)], e_v),
            ]

        def issue_loads(c, B):
            for s, d in load_list(c, B):
                pltpu.async_copy(s, d, B[7])

        def wait_loads(c, B):
            for s, d in load_list(c, B):
                pltpu.make_async_copy(s, d, B[7]).wait()

        def issue_gathers(B):
            pltpu.async_copy(nemb_hbm.at[B[3]], B[1], B[8])
            pltpu.async_copy(nemb_hbm.at[B[4]], B[2], B[9])

        def wait_gathers(B):
            pltpu.make_async_copy(nemb_hbm.at[B[3]], B[1], B[8]).wait()
            pltpu.make_async_copy(nemb_hbm.at[B[4]], B[2], B[9]).wait()

        def compute_scatter(B):
            e_v, nr_v, ns_v, ir_v, is_v, irp_v, isp_v = B[:7]

            def edge(i2, _):
                exs_l = []
                for u in range(8):
                    i = i2 * 8 + u
                    es = [e_v[i, pl.ds(j * LANES, LANES)] for j in range(DK)]
                    accr = es[0] * nr_v[i, pl.ds(0, LANES)]
                    accs = es[0] * ns_v[i, pl.ds(0, LANES)]
                    for j in range(1, DK):
                        accr = accr + es[j] * nr_v[i, pl.ds(j * LANES, LANES)]
                        accs = accs + es[j] * ns_v[i, pl.ds(j * LANES, LANES)]
                    exr = jnp.exp(_hsum(accr) * inv_scale_v)
                    exs = jnp.exp(_hsum(accs) * inv_scale_v)
                    for j in range(DK):
                        nr_v[i, pl.ds(j * LANES, LANES)] = es[j] * exr
                        ns_v[i, pl.ds(j * LANES, LANES)] = es[j] * exs
                    exs_l.append((i, exr, exs))
                for (i, exr, exs) in exs_l:
                    nri = irp_v[pl.ds(i, 16)][0]
                    den_v[pl.ds(nri, 16)] = den_v[pl.ds(nri, 16)] + exr * eye0
                    nsi = isp_v[pl.ds(i, 16)][0]
                    den_v[pl.ds(nsi, 16)] = den_v[pl.ds(nsi, 16)] + exs * eye0
                return 0

            lax.fori_loop(0, EK // 8, edge, 0)
            pltpu.sync_copy(nr_v, u_sh.at[ir_v], add=True)
            pltpu.sync_copy(ns_v, u_sh.at[is_v], add=True)

        issue_loads(0, bufs[0])
        wait_loads(0, bufs[0])
        issue_gathers(bufs[0])
        if NCH > 1:
            issue_loads(1, bufs[1])

        def pair(p, _):
            for bpar in (0, 1):
                c = 2 * p + bpar
                B = bufs[bpar]
                Bn = bufs[1 - bpar]

                @pl.when(c + 1 < NCH)
                def _():
                    wait_loads(c + 1, Bn)
                    issue_gathers(Bn)

                wait_gathers(B)
                compute_scatter(B)

                @pl.when(c + 2 < NCH)
                def _():
                    issue_loads(c + 2, B)
            return 0

        lax.fori_loop(0, NCH // 2, pair, 0)
        plsc.subcore_barrier()

        pltpu.sync_copy(u_sh.at[pl.ds(zoff, DPT)],
                        u_out.at[pl.ds(cid * N + zoff, DPT)])
        if DREM:
            @pl.when(sid == NS - 1)
            def _():
                pltpu.sync_copy(u_sh.at[pl.ds(NS * DPT, DREM)],
                                u_out.at[pl.ds(cid * N + NS * DPT, DREM)])
        pltpu.sync_copy(den_v.at[pl.ds(0, N)],
                        den_out.at[pl.ds(wid * NPAD, N)])

    return k1


@functools.lru_cache(maxsize=None)
def _norm_kernel(N, D, softmax_eps, many_parts):
    CH = 128
    FULLC = N // CH
    TREM = N - FULLC * CH
    TAILW = FULLC % NW
    ROUNDS = (FULLC + NW - 1) // NW
    DK = D // LANES
    NPAD = ((N + 127) // 128) * 128
    assert TREM % 16 == 0

    scratch = [
        pltpu.VMEM((CH, D), F32),
        pltpu.VMEM((CH, D), F32),
    ]
    if many_parts:
        scratch.append(pltpu.VMEM((NW, CH), F32))
    else:
        scratch += [pltpu.VMEM((CH,), F32), pltpu.VMEM((CH,), F32)]

    @functools.partial(
        pl.kernel,
        out_type=jax.ShapeDtypeStruct((N, D), F32),
        mesh=_mesh(),
        scratch_types=scratch,
    )
    def k(x_hbm, d_hbm, o_hbm, a_v, b_v, *dbufs):
        sid = lax.axis_index("s")
        cid = lax.axis_index("c")
        wid = sid * NC + cid
        iota = lax.iota(jnp.int32, LANES)
        onev = jnp.full((LANES,), 1.0, F32)
        epsv = jnp.full((LANES,), 1e-16, F32)

        def do(base, nrows):
            pltpu.sync_copy(x_hbm.at[pl.ds(base, nrows)], a_v.at[pl.ds(0, nrows)])
            pltpu.sync_copy(x_hbm.at[pl.ds(N + base, nrows)], b_v.at[pl.ds(0, nrows)])
            if many_parts:
                dbuf = dbufs[0]
                pltpu.sync_copy(d_hbm.at[:, pl.ds(base, CH)], dbuf)
            else:
                da_v, db_v = dbufs
                pltpu.sync_copy(d_hbm.at[pl.ds(base, nrows)], da_v.at[pl.ds(0, nrows)])
                pltpu.sync_copy(d_hbm.at[pl.ds(N + base, nrows)], db_v.at[pl.ds(0, nrows)])

            def group(g, _):
                if many_parts:
                    dbuf = dbufs[0]
                    d = dbuf[0, pl.ds(g * 16, 16)]
                    for w in range(1, NW):
                        d = d + dbuf[w, pl.ds(g * 16, 16)]
                else:
                    da_v, db_v = dbufs
                    d = da_v[pl.ds(g * 16, 16)] + db_v[pl.ds(g * 16, 16)]
                if softmax_eps:
                    inv16 = onev / (d + epsv)
                else:
                    inv16 = onev / jnp.maximum(d, onev)
                for l in range(LANES):
                    inv = inv16.at[iota * 0 + l].get(
                        mode="promise_in_bounds", unique_indices=False)
                    r = g * 16 + l
                    for j in range(DK):
                        sl = pl.ds(j * LANES, LANES)
                        a_v[r, sl] = (a_v[r, sl] + b_v[r, sl]) * inv
                return 0

            lax.fori_loop(0, nrows // 16, group, 0)
            pltpu.sync_copy(a_v.at[pl.ds(0, nrows)], o_hbm.at[pl.ds(base, nrows)])

        for t in range(ROUNDS):
            c = wid + NW * t
            if (t + 1) * NW <= FULLC:
                do(c * CH, CH)
            else:
                @pl.when(c < FULLC)
                def _():
                    do(c * CH, CH)
        if TREM:
            @pl.when(wid == TAILW)
            def _():
                do(FULLC * CH, TREM)

    return k


@functools.lru_cache(maxsize=None)
def _cell_gather_kernel(NCELL, N, D):
    CPW = (NCELL // NW) // 8 * 8
    REM = NCELL - CPW * NW
    CK = _chunk_1d(CPW)
    NCH = CPW // CK
    DK = D // LANES

    scratch = [
        pltpu.VMEM((CK, D), F32),
        pltpu.VMEM((CK, D), F32),
        pltpu.VMEM((CK, D), F32),
        pltpu.VMEM((CK, D), F32),
        pltpu.VMEM((CK,), jnp.int32),
        pltpu.VMEM((CK,), jnp.int32),
        pltpu.VMEM((CK,), jnp.int32),
        pltpu.SemaphoreType.DMA,
        pltpu.SemaphoreType.DMA,
        pltpu.SemaphoreType.DMA,
    ]
    if REM:
        scratch += [
            pltpu.VMEM((REM,), jnp.int32),
            pltpu.VMEM((REM,), jnp.int32),
            pltpu.VMEM((REM,), jnp.int32),
        ]

    @functools.partial(
        pl.kernel,
        out_type=jax.ShapeDtypeStruct((NCELL, D), F32),
        mesh=_mesh(),
        scratch_types=scratch,
    )
    def k(f0_hbm, f1_hbm, f2_hbm, nagg_hbm, o_hbm,
          g0_v, g1_v, g2_v, o_v, i0_v, i1_v, i2_v, s0, s1, s2, *tails):
        sid = lax.axis_index("s")
        cid = lax.axis_index("c")
        wid = sid * NC + cid
        third = jnp.full((LANES,), 1.0 / 3.0, F32)

        def do(b, n, i0, i1, i2):
            pltpu.sync_copy(f0_hbm.at[pl.ds(b, n)], i0)
            pltpu.sync_copy(f1_hbm.at[pl.ds(b, n)], i1)
            pltpu.sync_copy(f2_hbm.at[pl.ds(b, n)], i2)
            c0 = pltpu.async_copy(nagg_hbm.at[i0], g0_v.at[pl.ds(0, n)], s0)
            c1 = pltpu.async_copy(nagg_hbm.at[i1], g1_v.at[pl.ds(0, n)], s1)
            c2 = pltpu.async_copy(nagg_hbm.at[i2], g2_v.at[pl.ds(0, n)], s2)
            c0.wait()
            c1.wait()
            c2.wait()

            def cell(i, _):
                for j in range(DK):
                    sl = pl.ds(j * LANES, LANES)
                    o_v[i, sl] = (g0_v[i, sl] + g1_v[i, sl] + g2_v[i, sl]) * third
                return 0

            lax.fori_loop(0, n, cell, 0)
            pltpu.sync_copy(o_v.at[pl.ds(0, n)], o_hbm.at[pl.ds(b, n)])

        def chunk(c, _):
            do(wid * CPW + c * CK, CK, i0_v, i1_v, i2_v)
            return 0

        lax.fori_loop(0, NCH, chunk, 0)
        if REM:
            t0, t1, t2 = tails

            @pl.when(wid == NW - 1)
            def _():
                do(NW * CPW, REM, t0, t1, t2)

    return k


@functools.lru_cache(maxsize=None)
def _cell_scatter_kernel(NCELL, N, D):
    CPW = (NCELL // NW) // 8 * 8
    REM = NCELL - CPW * NW
    CK = _chunk_1d(CPW)
    NCH = CPW // CK
    DPT = (N // NS) // 8 * 8
    DREM = N - DPT * NS

    scratch = [
        pltpu.VMEM((CK, D), F32),
        pltpu.VMEM((CK,), jnp.int32),
        pltpu.VMEM((CK,), jnp.int32),
        pltpu.VMEM((CK,), jnp.int32),
        pltpu.VMEM((CK + 16,), jnp.int32),
        pltpu.VMEM((CK + 16,), jnp.int32),
        pltpu.VMEM((CK + 16,), jnp.int32),
        pltpu.VMEM((N + 16,), F32),
        pltpu.VMEM((DPT,), F32),
        pltpu.VMEM((DPT,), F32),
        pltpu.VMEM((16,), F32),
        pltpu.VMEM((16,), F32),
        pltpu.VMEM_SHARED((N, D), F32),
        pltpu.VMEM_SHARED((NS * N,), F32),
    ]
    if REM:
        scratch += [
            pltpu.VMEM((REM,), jnp.int32),
            pltpu.VMEM((REM,), jnp.int32),
            pltpu.VMEM((REM,), jnp.int32),
            pltpu.VMEM((REM + 16,), jnp.int32),
            pltpu.VMEM((REM + 16,), jnp.int32),
            pltpu.VMEM((REM + 16,), jnp.int32),
        ]

    @functools.partial(
        pl.kernel,
        out_type=(
            jax.ShapeDtypeStruct((NC * N, D), F32),
            jax.ShapeDtypeStruct((NC * N,), F32),
        ),
        mesh=_mesh(),
        scratch_types=scratch,
    )
    def k(cell_hbm, f0_hbm, f1_hbm, f2_hbm, zn_hbm, s_out, c_out,
          buf_v, i0_v, i1_v, i2_v, i0p_v, i1p_v, i2p_v, cnt_v,
          acc_v, tmp_v, ta_v, tt_v, s_sh, cnt_st, *tails):
        sid = lax.axis_index("s")
        cid = lax.axis_index("c")
        wid = sid * NC + cid
        iota = lax.iota(jnp.int32, LANES)
        zeros16 = jnp.zeros((LANES,), F32)
        onev = jnp.full((LANES,), 1.0, F32)
        eye0 = jnp.where(iota == 0, onev, zeros16)

        def zc(i, _):
            cnt_v[pl.ds(i * 16, 16)] = zeros16
            return 0

        lax.fori_loop(0, (N + 16) // 16, zc, 0)
        zoff = sid * DPT
        pltpu.sync_copy(zn_hbm.at[pl.ds(zoff, DPT)], s_sh.at[pl.ds(zoff, DPT)])
        if DREM:
            @pl.when(sid == NS - 1)
            def _():
                pltpu.sync_copy(zn_hbm.at[pl.ds(NS * DPT, DREM)],
                                s_sh.at[pl.ds(NS * DPT, DREM)])
        plsc.subcore_barrier()

        def do(b, n, i0, i1, i2, i0p, i1p, i2p):
            pltpu.sync_copy(cell_hbm.at[pl.ds(b, n)], buf_v.at[pl.ds(0, n)])
            pltpu.sync_copy(f0_hbm.at[pl.ds(b, n)], i0)
            pltpu.sync_copy(f1_hbm.at[pl.ds(b, n)], i1)
            pltpu.sync_copy(f2_hbm.at[pl.ds(b, n)], i2)
            pltpu.sync_copy(f0_hbm.at[pl.ds(b, n)], i0p.at[pl.ds(0, n)])
            pltpu.sync_copy(f1_hbm.at[pl.ds(b, n)], i1p.at[pl.ds(0, n)])
            pltpu.sync_copy(f2_hbm.at[pl.ds(b, n)], i2p.at[pl.ds(0, n)])
            pltpu.sync_copy(buf_v.at[pl.ds(0, n)], s_sh.at[i0], add=True)
            pltpu.sync_copy(buf_v.at[pl.ds(0, n)], s_sh.at[i1], add=True)
            pltpu.sync_copy(buf_v.at[pl.ds(0, n)], s_sh.at[i2], add=True)

            def cellcnt(i, _):
                for ipref in (i0p, i1p, i2p):
                    nn = ipref[pl.ds(i, 16)][0]
                    cnt_v[pl.ds(nn, 16)] = cnt_v[pl.ds(nn, 16)] + eye0
                return 0

            lax.fori_loop(0, n, cellcnt, 0)

        def chunk(c, _):
            do(wid * CPW + c * CK, CK, i0_v, i1_v, i2_v, i0p_v, i1p_v, i2p_v)
            return 0

        lax.fori_loop(0, NCH, chunk, 0)
        if REM:
            t0, t1, t2, t0p, t1p, t2p = tails

            @pl.when(wid == NW - 1)
            def _():
                do(NW * CPW, REM, t0, t1, t2, t0p, t1p, t2p)

        pltpu.sync_copy(cnt_v.at[pl.ds(0, N)], cnt_st.at[pl.ds(sid * N, N)])
        plsc.subcore_barrier()
        pltpu.sync_copy(s_sh.at[pl.ds(zoff, DPT)],
                        s_out.at[pl.ds(cid * N + zoff, DPT)])
        if DREM:
            @pl.when(sid == NS - 1)
            def _():
                pltpu.sync_copy(s_sh.at[pl.ds(NS * DPT, DREM)],
                                s_out.at[pl.ds(cid * N + NS * DPT, DREM)])
        _tile_reduce_dump(cnt_st, acc_v, tmp_v, ta_v, tt_v, c_out,
                          N, DPT, DREM, sid, cid)

    return k


def _matmul(cell_attr, cell_agg, w1, w2, b8):
    M, D = cell_attr.shape
    BM = 2000
    assert M % BM == 0

    def body(a_ref, g_ref, w1_ref, w2_ref, b_ref, o_ref):
        o_ref[...] = (
            jnp.dot(a_ref[...], w1_ref[...], preferred_element_type=F32)
            + jnp.dot(g_ref[...], w2_ref[...], preferred_element_type=F32)
            + b_ref[0:1, :]
        )

    return pl.pallas_call(
        body,
        grid=(M // BM,),
        in_specs=[
            pl.BlockSpec((BM, D), lambda i: (i, 0)),
            pl.BlockSpec((BM, D), lambda i: (i, 0)),
            pl.BlockSpec((D, D), lambda i: (0, 0)),
            pl.BlockSpec((D, D), lambda i: (0, 0)),
            pl.BlockSpec((8, D), lambda i: (0, 0)),
        ],
        out_specs=pl.BlockSpec((BM, D), lambda i: (i, 0)),
        out_shape=jax.ShapeDtypeStruct((M, D), F32),
    )(cell_attr, cell_agg, w1, w2, b8)


def kernel(cell_attr, edge_attr, node_embedding, edge_index, face, W, b):
    E, D = edge_attr.shape
    N = node_embedding.shape[0]
    NCELL = cell_attr.shape[0]

    senders = edge_index[0]
    receivers = edge_index[1]
    f0, f1, f2 = face[0], face[1], face[2]

    zn = jnp.zeros((N, D), F32)
    u_parts, den_parts = _edge_kernel(E, N, D)(
        edge_attr, senders, receivers, node_embedding, zn)
    npad = ((N + 127) // 128) * 128
    node_agg = _norm_kernel(N, D, True, True)(
        u_parts, den_parts.reshape(NW, npad))
    cell_agg = _cell_gather_kernel(NCELL, N, D)(f0, f1, f2, node_agg)
    b8 = jnp.broadcast_to(b.reshape(1, D), (8, D))
    cell_new = _matmul(cell_attr, cell_agg, W[:D], W[D:], b8)
    s_parts, c_parts = _cell_scatter_kernel(NCELL, N, D)(
        cell_new, f0, f1, f2, zn)
    node_attr = _norm_kernel(N, D, False, False)(s_parts, c_parts)
    return cell_new, node_attr
